# Initial kernel scaffold; baseline (speedup 1.0000x reference)
import jax
import jax.numpy as jnp
from jax.experimental import pallas as pl


def kernel(x, e, w1, b1, w2, b2, edge_index):
    raise NotImplementedError("write your pallas kernel here")


# speedup vs baseline: 2.7026x; 2.7026x over previous
import functools

import jax
import jax.numpy as jnp
from jax import lax
from jax.experimental import pallas as pl
from jax.experimental.pallas import tpu as pltpu
from jax.experimental.pallas import tpu_sc as plsc

N_NODES = 10000
N_EDGES = 160000
X_IN = 128
E_IN = 16
E_OUT = 16
X_OUT = 128

NC = 2
NS = 16
NW = NC * NS

E_PAD = 163840
EDGES_PER_W = E_PAD // NW
CHUNK = 128
CHUNKS_PER_W = EDGES_PER_W // CHUNK
N_PAD = 10240
SUM_BUF = N_PAD


def _tables_body(x_ref, ws_ref, wd_ref, ps_ref, pd_ref):
    xb = x_ref[...]
    ps_ref[...] = jnp.dot(xb, ws_ref[...], preferred_element_type=jnp.float32)
    pd_ref[...] = jnp.dot(xb, wd_ref[...], preferred_element_type=jnp.float32)


def _node_tables(x2d, w1s, w1d):
    blk = 1000
    grid = N_NODES // blk
    return pl.pallas_call(
        _tables_body,
        grid=(grid,),
        in_specs=[
            pl.BlockSpec((blk, X_IN), lambda i: (i, 0)),
            pl.BlockSpec((X_IN, E_OUT), lambda i: (0, 0)),
            pl.BlockSpec((X_IN, E_OUT), lambda i: (0, 0)),
        ],
        out_specs=[
            pl.BlockSpec((blk, E_OUT), lambda i: (i, 0)),
            pl.BlockSpec((blk, E_OUT), lambda i: (i, 0)),
        ],
        out_shape=[
            jax.ShapeDtypeStruct((N_NODES, E_OUT), jnp.float32),
            jax.ShapeDtypeStruct((N_NODES, E_OUT), jnp.float32),
        ],
    )(x2d, w1s, w1d)


def _pe_body(e_ref, we_ref, b1_ref, pe_ref):
    pe_ref[...] = (
        jnp.dot(e_ref[...], we_ref[...], preferred_element_type=jnp.float32)
        + b1_ref[...]
    )


def _edge_table(e_pad, w1e, b1):
    blk = 2048
    grid = E_PAD // blk
    return pl.pallas_call(
        _pe_body,
        grid=(grid,),
        in_specs=[
            pl.BlockSpec((blk, E_IN), lambda i: (i, 0)),
            pl.BlockSpec((E_IN, E_OUT), lambda i: (0, 0)),
            pl.BlockSpec((1, E_OUT), lambda i: (0, 0)),
        ],
        out_specs=pl.BlockSpec((blk, E_OUT), lambda i: (i, 0)),
        out_shape=jax.ShapeDtypeStruct((E_PAD, E_OUT), jnp.float32),
    )(e_pad, w1e, b1)


def _sc_body(ps_hbm, pd_hbm, pe_hbm, src_hbm, dst_hbm,
             oute_hbm, psum_hbm, pcnt_hbm,
             src_v, dst_v, psg_v, pdg_v, pe_v, oute_v, sums_v, cnt_v,
             sem1, sem2, sem3):
    wid = lax.axis_index("s") * NC + lax.axis_index("c")
    zero16 = jnp.zeros((16,), jnp.float32)

    def zbody(i, carry):
        sums_v[pl.ds(i * 16, 16)] = zero16
        cnt_v[pl.ds(i * 16, 16)] = zero16
        return carry

    lax.fori_loop(0, SUM_BUF // 16, zbody, 0)

    base0 = wid * EDGES_PER_W
    lanes = lax.iota(jnp.int32, 16)
    last_lane = lanes == 15
    ones16 = jnp.full((16,), 1.0, jnp.float32)

    def chunk(ci, carry):
        base = base0 + ci * CHUNK
        pltpu.sync_copy(src_hbm.at[pl.ds(base, CHUNK)], src_v)
        pltpu.sync_copy(dst_hbm.at[pl.ds(base, CHUNK)], dst_v)
        cp1 = pltpu.async_copy(ps_hbm.at[src_v], psg_v, sem1)
        cp2 = pltpu.async_copy(pd_hbm.at[dst_v], pdg_v, sem2)
        cp3 = pltpu.async_copy(pe_hbm.at[pl.ds(base, CHUNK)], pe_v, sem3)
        cp1.wait()
        cp2.wait()
        cp3.wait()

        def row(r, rcarry):
            v = pe_v[r] + psg_v[r] + pdg_v[r]
            oute_v[pl.ds(r * E_OUT, 16)] = jnp.maximum(v, 0.0)
            return rcarry

        lax.fori_loop(0, CHUNK, row, 0, unroll=4)

        def grp(g, gcarry):
            flat = (lanes + g * 16) * E_OUT
            m = plsc.load_gather(oute_v, [flat])
            for c in range(1, E_OUT):
                m = m + plsc.load_gather(oute_v, [flat + c])
            dv = dst_v[pl.ds(g * 16, 16)]
            plsc.addupdate_scatter(sums_v, [dv], m)
            plsc.addupdate_scatter(cnt_v, [dv], ones16)
            return gcarry

        lax.fori_loop(0, CHUNK // 16, grp, 0, unroll=2)
        pltpu.sync_copy(oute_v, oute_hbm.at[pl.ds(base * E_OUT, CHUNK * E_OUT)])
        return carry

    lax.fori_loop(0, CHUNKS_PER_W, chunk, 0)
    pltpu.sync_copy(sums_v, psum_hbm.at[wid])
    pltpu.sync_copy(cnt_v, pcnt_hbm.at[wid])


_sc_edges = functools.partial(
    pl.kernel,
    out_type=[
        jax.ShapeDtypeStruct((E_PAD * E_OUT,), jnp.float32),
        jax.ShapeDtypeStruct((NW, N_PAD), jnp.float32),
        jax.ShapeDtypeStruct((NW, N_PAD), jnp.float32),
    ],
    mesh=plsc.VectorSubcoreMesh(core_axis_name="c", subcore_axis_name="s"),
    compiler_params=pltpu.CompilerParams(
        use_tc_tiling_on_sc=False, needs_layout_passes=False),
    scratch_types=[
        pltpu.VMEM((CHUNK,), jnp.int32),
        pltpu.VMEM((CHUNK,), jnp.int32),
        pltpu.VMEM((CHUNK, E_OUT), jnp.float32),
        pltpu.VMEM((CHUNK, E_OUT), jnp.float32),
        pltpu.VMEM((CHUNK, E_OUT), jnp.float32),
        pltpu.VMEM((CHUNK * E_OUT,), jnp.float32),
        pltpu.VMEM((SUM_BUF,), jnp.float32),
        pltpu.VMEM((SUM_BUF,), jnp.float32),
        pltpu.SemaphoreType.DMA,
        pltpu.SemaphoreType.DMA,
        pltpu.SemaphoreType.DMA,
    ],
)(_sc_body)


def _newx_body(x_ref, psum_ref, pcnt_ref, w2a_ref, w2b_ref, b2_ref, out_ref):
    s = jnp.sum(psum_ref[...], axis=0) * (1.0 / E_OUT)
    c = jnp.sum(pcnt_ref[...], axis=0)
    attr = s / jnp.maximum(c, 1.0)
    acc = jnp.dot(x_ref[...], w2a_ref[...], preferred_element_type=jnp.float32)
    acc = acc + attr[:, None] * w2b_ref[...] + b2_ref[...]
    out_ref[...] = jnp.maximum(acc, 0.0)


def _node_update(x_pad, psum, pcnt, w2a, w2b, b2):
    blk = 1024
    grid = N_PAD // blk
    return pl.pallas_call(
        _newx_body,
        grid=(grid,),
        in_specs=[
            pl.BlockSpec((blk, X_IN), lambda i: (i, 0)),
            pl.BlockSpec((NW, blk), lambda i: (0, i)),
            pl.BlockSpec((NW, blk), lambda i: (0, i)),
            pl.BlockSpec((X_IN, X_OUT), lambda i: (0, 0)),
            pl.BlockSpec((1, X_OUT), lambda i: (0, 0)),
            pl.BlockSpec((1, X_OUT), lambda i: (0, 0)),
        ],
        out_specs=pl.BlockSpec((blk, X_OUT), lambda i: (i, 0)),
        out_shape=jax.ShapeDtypeStruct((N_PAD, X_OUT), jnp.float32),
    )(x_pad, psum, pcnt, w2a, w2b, b2)


def kernel(x, e, w1, b1, w2, b2, edge_index):
    x2d = x[0]
    e2d = e[0]
    src = edge_index[0]
    dst = edge_index[1]

    w1e = w1[:E_IN]
    w1s = w1[E_IN:E_IN + X_IN]
    w1d = w1[E_IN + X_IN:]
    b1r = b1.reshape(1, E_OUT)
    w2a = w2[:X_IN]
    w2b = w2[X_IN:].reshape(1, X_OUT)
    b2r = b2.reshape(1, X_OUT)

    pad = E_PAD - N_EDGES
    e_pad = jnp.concatenate(
        [e2d, jnp.zeros((pad, E_IN), jnp.float32)], axis=0)
    src_pad = jnp.concatenate([src, jnp.zeros((pad,), jnp.int32)])
    dst_pad = jnp.concatenate(
        [dst, jnp.full((pad,), N_NODES, jnp.int32)])

    x_pad = jnp.concatenate(
        [x2d, jnp.zeros((N_PAD - N_NODES, X_IN), jnp.float32)], axis=0)

    ps, pd = _node_tables(x2d, w1s, w1d)
    pe = _edge_table(e_pad, w1e, b1r)
    new_e_flat, psum, pcnt = _sc_edges(ps, pd, pe, src_pad, dst_pad)
    new_x_pad = _node_update(x_pad, psum, pcnt, w2a, w2b, b2r)

    new_e = new_e_flat.reshape(E_PAD, E_OUT)[:N_EDGES]
    return new_x_pad[:N_NODES][None], new_e[None]


# speedup vs baseline: 3.8629x; 1.4293x over previous
import functools

import jax
import jax.numpy as jnp
from jax import lax
from jax.experimental import pallas as pl
from jax.experimental.pallas import tpu as pltpu
from jax.experimental.pallas import tpu_sc as plsc

N_NODES = 10000
N_EDGES = 160000
X_IN = 128
E_IN = 16
E_OUT = 16
X_OUT = 128

NC = 2
NS = 16
NW = NC * NS

CHUNK = 128
N_CHUNKS = N_EDGES // CHUNK
ITERS = -(-N_CHUNKS // NW)
N_PAD = 10240
SUM_BUF = N_PAD
OUT_BYTES = CHUNK * E_OUT * 4


def _tables_body(x_ref, ws_ref, wd_ref, ps_ref, pd_ref):
    xb = x_ref[...]
    ps_ref[...] = jnp.dot(xb, ws_ref[...], preferred_element_type=jnp.float32)
    pd_ref[...] = jnp.dot(xb, wd_ref[...], preferred_element_type=jnp.float32)


def _node_tables(x2d, w1s, w1d):
    blk = 1000
    grid = N_NODES // blk
    return pl.pallas_call(
        _tables_body,
        grid=(grid,),
        in_specs=[
            pl.BlockSpec((blk, X_IN), lambda i: (i, 0)),
            pl.BlockSpec((X_IN, E_OUT), lambda i: (0, 0)),
            pl.BlockSpec((X_IN, E_OUT), lambda i: (0, 0)),
        ],
        out_specs=[
            pl.BlockSpec((blk, E_OUT), lambda i: (i, 0)),
            pl.BlockSpec((blk, E_OUT), lambda i: (i, 0)),
        ],
        out_shape=[
            jax.ShapeDtypeStruct((N_NODES, E_OUT), jnp.float32),
            jax.ShapeDtypeStruct((N_NODES, E_OUT), jnp.float32),
        ],
    )(x2d, w1s, w1d)


def _pe_body(e_ref, we_ref, b1_ref, pe_ref):
    pe_ref[...] = (
        jnp.dot(e_ref[...], we_ref[...], preferred_element_type=jnp.float32)
        + b1_ref[...]
    )


def _edge_table(e2d, w1e, b1):
    blk = 2048
    grid = -(-N_EDGES // blk)
    return pl.pallas_call(
        _pe_body,
        grid=(grid,),
        in_specs=[
            pl.BlockSpec((blk, E_IN), lambda i: (i, 0)),
            pl.BlockSpec((E_IN, E_OUT), lambda i: (0, 0)),
            pl.BlockSpec((1, E_OUT), lambda i: (0, 0)),
        ],
        out_specs=pl.BlockSpec((blk, E_OUT), lambda i: (i, 0)),
        out_shape=jax.ShapeDtypeStruct((N_EDGES, E_OUT), jnp.float32),
    )(e2d, w1e, b1)


def _sc_body(ps_hbm, pd_hbm, pe_hbm, src_hbm, dst_hbm,
             oute_hbm, psum_hbm, pcnt_hbm,
             srcv, dstv, psgv, pdgv, pev, outv, sums_v, cnt_v,
             semi0, semi1, semg0, semg1, semo0, semo1):
    wid = lax.axis_index("s") * NC + lax.axis_index("c")
    semi = (semi0, semi1)
    semg = (semg0, semg1)
    semo = (semo0, semo1)
    zero16 = jnp.zeros((16,), jnp.float32)
    lanes = lax.iota(jnp.int32, 16)
    ones16 = jnp.full((16,), 1.0, jnp.float32)

    def zbody(i, carry):
        sums_v[pl.ds(i * 16, 16)] = zero16
        cnt_v[pl.ds(i * 16, 16)] = zero16
        return carry

    lax.fori_loop(0, SUM_BUF // 16, zbody, 0)

    def gbase(ci):
        g = ci * NW + wid
        return jnp.minimum(g, N_CHUNKS - 1) * CHUNK

    def load_idx(ci, b):
        base = gbase(ci)
        pltpu.async_copy(src_hbm.at[pl.ds(base, CHUNK)], srcv.at[b], semi[b])
        pltpu.async_copy(dst_hbm.at[pl.ds(base, CHUNK)], dstv.at[b], semi[b])

    def wait_idx(b):
        pltpu.make_async_copy(
            src_hbm.at[pl.ds(0, CHUNK)], srcv.at[b], semi[b]).wait()
        pltpu.make_async_copy(
            dst_hbm.at[pl.ds(0, CHUNK)], dstv.at[b], semi[b]).wait()

    def start_gathers(ci, b):
        base = gbase(ci)
        pltpu.async_copy(ps_hbm.at[srcv.at[b]], psgv.at[b], semg[b])
        pltpu.async_copy(pd_hbm.at[dstv.at[b]], pdgv.at[b], semg[b])
        pltpu.async_copy(pe_hbm.at[pl.ds(base, CHUNK)], pev.at[b], semg[b])

    def wait_gathers(b):
        pltpu.make_async_copy(
            pe_hbm.at[pl.ds(0, CHUNK)], psgv.at[b], semg[b]).wait()
        pltpu.make_async_copy(
            pe_hbm.at[pl.ds(0, CHUNK)], pdgv.at[b], semg[b]).wait()
        pltpu.make_async_copy(
            pe_hbm.at[pl.ds(0, CHUNK)], pev.at[b], semg[b]).wait()

    def wait_out(b):
        pltpu.make_async_copy(
            oute_hbm.at[pl.ds(0, CHUNK * E_OUT)], outv.at[b], semo[b]).wait()

    load_idx(0, 0)
    wait_idx(0)
    load_idx(1, 1)
    start_gathers(0, 0)

    def it_body(it, carry):
        for b in (0, 1):
            ci = 2 * it + b
            b1 = 1 - b
            wait_idx(b1)
            start_gathers(ci + 1, b1)
            wait_gathers(b)
            @pl.when(ci >= 2)
            def _():
                wait_out(b)

            def row(r, rcarry):
                v = pev.at[b][r] + psgv.at[b][r] + pdgv.at[b][r]
                outv.at[b][pl.ds(r * E_OUT, 16)] = jnp.maximum(v, 0.0)
                return rcarry

            lax.fori_loop(0, CHUNK, row, 0, unroll=4)

            valid = ((ci * NW + wid) < N_CHUNKS).astype(jnp.float32)
            vf = jnp.broadcast_to(valid, (16,))

            def grp(g, gcarry):
                flat = (lanes + g * 16) * E_OUT
                m = plsc.load_gather(outv.at[b], [flat])
                for c in range(1, E_OUT):
                    m = m + plsc.load_gather(outv.at[b], [flat + c])
                dv = dstv.at[b][pl.ds(g * 16, 16)]
                plsc.addupdate_scatter(sums_v, [dv], m * vf)
                plsc.addupdate_scatter(cnt_v, [dv], vf)
                return gcarry

            lax.fori_loop(0, CHUNK // 16, grp, 0, unroll=2)
            pltpu.async_copy(
                outv.at[b],
                oute_hbm.at[pl.ds(gbase(ci) * E_OUT, CHUNK * E_OUT)],
                semo[b])
            load_idx(ci + 2, b)
        return carry

    lax.fori_loop(0, ITERS // 2, it_body, 0)

    wait_gathers(0)
    wait_idx(1)
    wait_out(0)
    wait_out(1)

    pltpu.sync_copy(sums_v, psum_hbm.at[wid])
    pltpu.sync_copy(cnt_v, pcnt_hbm.at[wid])


_sc_edges = functools.partial(
    pl.kernel,
    out_type=[
        jax.ShapeDtypeStruct((N_EDGES * E_OUT,), jnp.float32),
        jax.ShapeDtypeStruct((NW, N_PAD), jnp.float32),
        jax.ShapeDtypeStruct((NW, N_PAD), jnp.float32),
    ],
    mesh=plsc.VectorSubcoreMesh(core_axis_name="c", subcore_axis_name="s"),
    compiler_params=pltpu.CompilerParams(
        use_tc_tiling_on_sc=False, needs_layout_passes=False),
    scratch_types=[
        pltpu.VMEM((2, CHUNK), jnp.int32),
        pltpu.VMEM((2, CHUNK), jnp.int32),
        pltpu.VMEM((2, CHUNK, E_OUT), jnp.float32),
        pltpu.VMEM((2, CHUNK, E_OUT), jnp.float32),
        pltpu.VMEM((2, CHUNK, E_OUT), jnp.float32),
        pltpu.VMEM((2, CHUNK * E_OUT), jnp.float32),
        pltpu.VMEM((SUM_BUF,), jnp.float32),
        pltpu.VMEM((SUM_BUF,), jnp.float32),
        pltpu.SemaphoreType.DMA,
        pltpu.SemaphoreType.DMA,
        pltpu.SemaphoreType.DMA,
        pltpu.SemaphoreType.DMA,
        pltpu.SemaphoreType.DMA,
        pltpu.SemaphoreType.DMA,
    ],
)(_sc_body)


def _newx_body(x_ref, psum_ref, pcnt_ref, w2a_ref, w2b_ref, b2_ref, out_ref):
    s = jnp.sum(psum_ref[...], axis=0) * (1.0 / E_OUT)
    c = jnp.sum(pcnt_ref[...], axis=0)
    attr = s / jnp.maximum(c, 1.0)
    acc = jnp.dot(x_ref[...], w2a_ref[...], preferred_element_type=jnp.float32)
    acc = acc + attr[:, None] * w2b_ref[...] + b2_ref[...]
    out_ref[...] = jnp.maximum(acc, 0.0)


def _node_update(x2d, psum, pcnt, w2a, w2b, b2):
    blk = 1024
    grid = N_PAD // blk
    return pl.pallas_call(
        _newx_body,
        grid=(grid,),
        in_specs=[
            pl.BlockSpec((blk, X_IN), lambda i: (i, 0)),
            pl.BlockSpec((NW, blk), lambda i: (0, i)),
            pl.BlockSpec((NW, blk), lambda i: (0, i)),
            pl.BlockSpec((X_IN, X_OUT), lambda i: (0, 0)),
            pl.BlockSpec((1, X_OUT), lambda i: (0, 0)),
            pl.BlockSpec((1, X_OUT), lambda i: (0, 0)),
        ],
        out_specs=pl.BlockSpec((blk, X_OUT), lambda i: (i, 0)),
        out_shape=jax.ShapeDtypeStruct((N_NODES, X_OUT), jnp.float32),
    )(x2d, psum, pcnt, w2a, w2b, b2)


def kernel(x, e, w1, b1, w2, b2, edge_index):
    x2d = x[0]
    e2d = e[0]
    src = edge_index[0]
    dst = edge_index[1]

    w1e = w1[:E_IN]
    w1s = w1[E_IN:E_IN + X_IN]
    w1d = w1[E_IN + X_IN:]
    b1r = b1.reshape(1, E_OUT)
    w2a = w2[:X_IN]
    w2b = w2[X_IN:].reshape(1, X_OUT)
    b2r = b2.reshape(1, X_OUT)

    ps, pd = _node_tables(x2d, w1s, w1d)
    pe = _edge_table(e2d, w1e, b1r)
    new_e_flat, psum, pcnt = _sc_edges(ps, pd, pe, src, dst)
    new_x = _node_update(x2d, psum, pcnt, w2a, w2b, b2r)

    return new_x[None], new_e_flat.reshape(N_EDGES, E_OUT)[None]


# speedup vs baseline: 4.9219x; 1.2741x over previous
import functools

import jax
import jax.numpy as jnp
from jax import lax
from jax.experimental import pallas as pl
from jax.experimental.pallas import tpu as pltpu
from jax.experimental.pallas import tpu_sc as plsc

N_NODES = 10000
N_EDGES = 160000
X_IN = 128
E_IN = 16
E_OUT = 16
X_OUT = 128

NC = 2
NS = 16
NW = NC * NS

CHUNK = 128
N_CHUNKS = N_EDGES // CHUNK
ITERS = -(-N_CHUNKS // NW)
N_PAD = 10240
SUM_BUF = N_PAD
OUT_BYTES = CHUNK * E_OUT * 4


def _tables_body(x_ref, ws_ref, wd_ref, ps_ref, pd_ref):
    xb = x_ref[...]
    ps_ref[...] = jnp.dot(xb, ws_ref[...], preferred_element_type=jnp.float32)
    pd_ref[...] = jnp.dot(xb, wd_ref[...], preferred_element_type=jnp.float32)


def _node_tables(x2d, w1s, w1d):
    blk = 1000
    grid = N_NODES // blk
    return pl.pallas_call(
        _tables_body,
        grid=(grid,),
        in_specs=[
            pl.BlockSpec((blk, X_IN), lambda i: (i, 0)),
            pl.BlockSpec((X_IN, E_OUT), lambda i: (0, 0)),
            pl.BlockSpec((X_IN, E_OUT), lambda i: (0, 0)),
        ],
        out_specs=[
            pl.BlockSpec((blk, E_OUT), lambda i: (i, 0)),
            pl.BlockSpec((blk, E_OUT), lambda i: (i, 0)),
        ],
        out_shape=[
            jax.ShapeDtypeStruct((N_NODES, E_OUT), jnp.float32),
            jax.ShapeDtypeStruct((N_NODES, E_OUT), jnp.float32),
        ],
    )(x2d, w1s, w1d)


def _pe_body(e_ref, we_ref, b1_ref, pe_ref):
    pe_ref[...] = (
        jnp.dot(e_ref[...], we_ref[...], preferred_element_type=jnp.float32)
        + b1_ref[...]
    )


def _edge_table(e8, w8, b18):
    blk = 2048
    grid = N_EDGES // 8 // blk
    return pl.pallas_call(
        _pe_body,
        grid=(grid,),
        in_specs=[
            pl.BlockSpec((blk, 128), lambda i: (i, 0)),
            pl.BlockSpec((128, 128), lambda i: (0, 0)),
            pl.BlockSpec((1, 128), lambda i: (0, 0)),
        ],
        out_specs=pl.BlockSpec((blk, 128), lambda i: (i, 0)),
        out_shape=jax.ShapeDtypeStruct((N_EDGES // 8, 128), jnp.float32),
    )(e8, w8, b18)


def _sc_body(ps_hbm, pd_hbm, pe_hbm, src_hbm, dst_hbm,
             oute_hbm, psum_hbm, pcnt_hbm,
             srcv, dstv, psgv, pdgv, pev, outv, sums_v, cnt_v,
             semi0, semi1, semg0, semg1, semo0, semo1):
    wid = lax.axis_index("s") * NC + lax.axis_index("c")
    semi = (semi0, semi1)
    semg = (semg0, semg1)
    semo = (semo0, semo1)
    zero16 = jnp.zeros((16,), jnp.float32)
    lanes = lax.iota(jnp.int32, 16)
    ones16 = jnp.full((16,), 1.0, jnp.float32)

    def zbody(i, carry):
        sums_v[pl.ds(i * 16, 16)] = zero16
        cnt_v[pl.ds(i * 16, 16)] = zero16
        return carry

    lax.fori_loop(0, SUM_BUF // 16, zbody, 0)

    def gchunk(ci):
        g = ci * NW + wid
        return jnp.minimum(g, N_CHUNKS - 1)

    def gbase(ci):
        return gchunk(ci) * CHUNK

    def load_idx(ci, b):
        base = gbase(ci)
        pltpu.async_copy(src_hbm.at[pl.ds(base, CHUNK)], srcv.at[b], semi[b])
        pltpu.async_copy(dst_hbm.at[pl.ds(base, CHUNK)], dstv.at[b], semi[b])

    def wait_idx(b):
        pltpu.make_async_copy(
            src_hbm.at[pl.ds(0, CHUNK)], srcv.at[b], semi[b]).wait()
        pltpu.make_async_copy(
            dst_hbm.at[pl.ds(0, CHUNK)], dstv.at[b], semi[b]).wait()

    def start_gathers(ci, b):
        base = gbase(ci)
        pltpu.async_copy(ps_hbm.at[srcv.at[b]], psgv.at[b], semg[b])
        pltpu.async_copy(pd_hbm.at[dstv.at[b]], pdgv.at[b], semg[b])
        pltpu.async_copy(
            pe_hbm.at[pl.ds(gchunk(ci) * (CHUNK // 8), CHUNK // 8)],
            pev.at[b], semg[b])

    def wait_gathers(b):
        pltpu.make_async_copy(
            ps_hbm.at[pl.ds(0, CHUNK)], psgv.at[b], semg[b]).wait()
        pltpu.make_async_copy(
            ps_hbm.at[pl.ds(0, CHUNK)], pdgv.at[b], semg[b]).wait()
        pltpu.make_async_copy(
            pe_hbm.at[pl.ds(0, CHUNK // 8)], pev.at[b], semg[b]).wait()

    def wait_out(b):
        pltpu.make_async_copy(
            oute_hbm.at[pl.ds(0, CHUNK * E_OUT)], outv.at[b], semo[b]).wait()

    load_idx(0, 0)
    wait_idx(0)
    load_idx(1, 1)
    start_gathers(0, 0)

    def it_body(it, carry):
        for b in (0, 1):
            ci = 2 * it + b
            b1 = 1 - b
            wait_idx(b1)
            start_gathers(ci + 1, b1)
            wait_gathers(b)
            @pl.when(ci >= 2)
            def _():
                wait_out(b)

            def row(r, rcarry):
                pe_row = pev.at[b][
                    lax.shift_right_logical(r, 3),
                    pl.ds(lax.mul(lax.bitwise_and(r, 7), E_OUT), 16)]
                v = pe_row + psgv.at[b][r] + pdgv.at[b][r]
                outv.at[b][pl.ds(r * E_OUT, 16)] = jnp.maximum(v, 0.0)
                return rcarry

            lax.fori_loop(0, CHUNK, row, 0, unroll=4)

            valid = ((ci * NW + wid) < N_CHUNKS).astype(jnp.float32)
            vf = jnp.broadcast_to(valid, (16,))

            def grp(g, gcarry):
                flat = (lanes + g * 16) * E_OUT
                m = plsc.load_gather(outv.at[b], [flat])
                for c in range(1, E_OUT):
                    m = m + plsc.load_gather(outv.at[b], [flat + c])
                dv = dstv.at[b][pl.ds(g * 16, 16)]
                plsc.addupdate_scatter(sums_v, [dv], m * vf)
                plsc.addupdate_scatter(cnt_v, [dv], vf)
                return gcarry

            lax.fori_loop(0, CHUNK // 16, grp, 0, unroll=2)
            pltpu.async_copy(
                outv.at[b],
                oute_hbm.at[pl.ds(gbase(ci) * E_OUT, CHUNK * E_OUT)],
                semo[b])
            load_idx(ci + 2, b)
        return carry

    lax.fori_loop(0, ITERS // 2, it_body, 0)

    wait_gathers(0)
    wait_idx(1)
    wait_out(0)
    wait_out(1)

    pltpu.sync_copy(sums_v, psum_hbm.at[wid])
    pltpu.sync_copy(cnt_v, pcnt_hbm.at[wid])


_sc_edges = functools.partial(
    pl.kernel,
    out_type=[
        jax.ShapeDtypeStruct((N_EDGES * E_OUT,), jnp.float32),
        jax.ShapeDtypeStruct((NW, N_PAD), jnp.float32),
        jax.ShapeDtypeStruct((NW, N_PAD), jnp.float32),
    ],
    mesh=plsc.VectorSubcoreMesh(core_axis_name="c", subcore_axis_name="s"),
    compiler_params=pltpu.CompilerParams(
        use_tc_tiling_on_sc=False, needs_layout_passes=False),
    scratch_types=[
        pltpu.VMEM((2, CHUNK), jnp.int32),
        pltpu.VMEM((2, CHUNK), jnp.int32),
        pltpu.VMEM((2, CHUNK, E_OUT), jnp.float32),
        pltpu.VMEM((2, CHUNK, E_OUT), jnp.float32),
        pltpu.VMEM((2, CHUNK // 8, 128), jnp.float32),
        pltpu.VMEM((2, CHUNK * E_OUT), jnp.float32),
        pltpu.VMEM((SUM_BUF,), jnp.float32),
        pltpu.VMEM((SUM_BUF,), jnp.float32),
        pltpu.SemaphoreType.DMA,
        pltpu.SemaphoreType.DMA,
        pltpu.SemaphoreType.DMA,
        pltpu.SemaphoreType.DMA,
        pltpu.SemaphoreType.DMA,
        pltpu.SemaphoreType.DMA,
    ],
)(_sc_body)


def _newx_body(x_ref, psum_ref, pcnt_ref, w2a_ref, w2b_ref, b2_ref, out_ref):
    s = jnp.sum(psum_ref[...], axis=0) * (1.0 / E_OUT)
    c = jnp.sum(pcnt_ref[...], axis=0)
    attr = s / jnp.maximum(c, 1.0)
    acc = jnp.dot(x_ref[...], w2a_ref[...], preferred_element_type=jnp.float32)
    acc = acc + attr[:, None] * w2b_ref[...] + b2_ref[...]
    out_ref[...] = jnp.maximum(acc, 0.0)


def _node_update(x2d, psum, pcnt, w2a, w2b, b2):
    blk = 1024
    grid = N_PAD // blk
    return pl.pallas_call(
        _newx_body,
        grid=(grid,),
        in_specs=[
            pl.BlockSpec((blk, X_IN), lambda i: (i, 0)),
            pl.BlockSpec((NW, blk), lambda i: (0, i)),
            pl.BlockSpec((NW, blk), lambda i: (0, i)),
            pl.BlockSpec((X_IN, X_OUT), lambda i: (0, 0)),
            pl.BlockSpec((1, X_OUT), lambda i: (0, 0)),
            pl.BlockSpec((1, X_OUT), lambda i: (0, 0)),
        ],
        out_specs=pl.BlockSpec((blk, X_OUT), lambda i: (i, 0)),
        out_shape=jax.ShapeDtypeStruct((N_NODES, X_OUT), jnp.float32),
    )(x2d, psum, pcnt, w2a, w2b, b2)


def kernel(x, e, w1, b1, w2, b2, edge_index):
    x2d = x[0]
    e2d = e[0]
    src = edge_index[0]
    dst = edge_index[1]

    w1e = w1[:E_IN]
    w1s = w1[E_IN:E_IN + X_IN]
    w1d = w1[E_IN + X_IN:]
    b1r = b1.reshape(1, E_OUT)
    w2a = w2[:X_IN]
    w2b = w2[X_IN:].reshape(1, X_OUT)
    b2r = b2.reshape(1, X_OUT)

    e8 = e.reshape(N_EDGES // 8, 8 * E_IN)
    w8 = jnp.kron(jnp.eye(8, dtype=jnp.float32), w1e)
    b18 = jnp.tile(b1, 8).reshape(1, 8 * E_OUT)

    ps, pd = _node_tables(x2d, w1s, w1d)
    pe = _edge_table(e8, w8, b18)
    new_e_flat, psum, pcnt = _sc_edges(ps, pd, pe, src, dst)
    new_x = _node_update(x2d, psum, pcnt, w2a, w2b, b2r)

    return new_x[None], new_e_flat.reshape(N_EDGES, E_OUT)[None]


# speedup vs baseline: 4.9258x; 1.0008x over previous
import functools

import jax
import jax.numpy as jnp
from jax import lax
from jax.experimental import pallas as pl
from jax.experimental.pallas import tpu as pltpu
from jax.experimental.pallas import tpu_sc as plsc

N_NODES = 10000
N_EDGES = 160000
X_IN = 128
E_IN = 16
E_OUT = 16
X_OUT = 128

NC = 2
NS = 16
NW = NC * NS

CHUNK = 128
N_CHUNKS = N_EDGES // CHUNK
ITERS = -(-N_CHUNKS // NW)
N_PAD = 10240
SUM_BUF = N_PAD
OUT_BYTES = CHUNK * E_OUT * 4


def _tables_body(x_ref, ws_ref, wd_ref, ps_ref, pd_ref):
    xb = x_ref[...]
    ps_ref[...] = jnp.dot(xb, ws_ref[...], preferred_element_type=jnp.float32)
    pd_ref[...] = jnp.dot(xb, wd_ref[...], preferred_element_type=jnp.float32)


def _node_tables(x2d, w1s, w1d):
    blk = 1000
    grid = N_NODES // blk
    return pl.pallas_call(
        _tables_body,
        grid=(grid,),
        in_specs=[
            pl.BlockSpec((blk, X_IN), lambda i: (i, 0)),
            pl.BlockSpec((X_IN, E_OUT), lambda i: (0, 0)),
            pl.BlockSpec((X_IN, E_OUT), lambda i: (0, 0)),
        ],
        out_specs=[
            pl.BlockSpec((blk, E_OUT), lambda i: (i, 0)),
            pl.BlockSpec((blk, E_OUT), lambda i: (i, 0)),
        ],
        out_shape=[
            jax.ShapeDtypeStruct((N_NODES, E_OUT), jnp.float32),
            jax.ShapeDtypeStruct((N_NODES, E_OUT), jnp.float32),
        ],
    )(x2d, w1s, w1d)


def _pe_body(e_ref, we_ref, b1_ref, pe_ref):
    pe_ref[...] = (
        jnp.dot(e_ref[...], we_ref[...], preferred_element_type=jnp.float32)
        + b1_ref[...]
    )


def _edge_table(e8, w8, b18):
    blk = 2000
    grid = N_EDGES // 8 // blk
    return pl.pallas_call(
        _pe_body,
        grid=(grid,),
        in_specs=[
            pl.BlockSpec((blk, 128), lambda i: (i, 0)),
            pl.BlockSpec((128, 128), lambda i: (0, 0)),
            pl.BlockSpec((1, 128), lambda i: (0, 0)),
        ],
        out_specs=pl.BlockSpec((blk, 128), lambda i: (i, 0)),
        out_shape=jax.ShapeDtypeStruct((N_EDGES // 8, 128), jnp.float32),
    )(e8, w8, b18)


def _sc_body(ps_hbm, pd_hbm, pe_hbm, src_hbm, dst_hbm,
             oute_hbm, psum_hbm, pcnt_hbm,
             srcv, dstv, psgv, pdgv, pev, outv, sums_v, cnt_v,
             semi0, semi1, semg0, semg1, semo0, semo1):
    wid = lax.axis_index("s") * NC + lax.axis_index("c")
    semi = (semi0, semi1)
    semg = (semg0, semg1)
    semo = (semo0, semo1)
    zero16 = jnp.zeros((16,), jnp.float32)
    lanes = lax.iota(jnp.int32, 16)
    ones16 = jnp.full((16,), 1.0, jnp.float32)

    def zbody(i, carry):
        sums_v[pl.ds(i * 16, 16)] = zero16
        cnt_v[pl.ds(i * 16, 16)] = zero16
        return carry

    lax.fori_loop(0, SUM_BUF // 16, zbody, 0)

    def gchunk(ci):
        g = ci * NW + wid
        return jnp.minimum(g, N_CHUNKS - 1)

    def gbase(ci):
        return gchunk(ci) * CHUNK

    def load_idx(ci, b):
        base = gbase(ci)
        pltpu.async_copy(src_hbm.at[pl.ds(base, CHUNK)], srcv.at[b], semi[b])
        pltpu.async_copy(dst_hbm.at[pl.ds(base, CHUNK)], dstv.at[b], semi[b])

    def wait_idx(b):
        pltpu.make_async_copy(
            src_hbm.at[pl.ds(0, CHUNK)], srcv.at[b], semi[b]).wait()
        pltpu.make_async_copy(
            dst_hbm.at[pl.ds(0, CHUNK)], dstv.at[b], semi[b]).wait()

    def start_gathers(ci, b):
        base = gbase(ci)
        pltpu.async_copy(ps_hbm.at[srcv.at[b]], psgv.at[b], semg[b])
        pltpu.async_copy(pd_hbm.at[dstv.at[b]], pdgv.at[b], semg[b])
        pltpu.async_copy(
            pe_hbm.at[pl.ds(gchunk(ci) * (CHUNK // 8), CHUNK // 8)],
            pev.at[b], semg[b])

    def wait_gathers(b):
        pltpu.make_async_copy(
            ps_hbm.at[pl.ds(0, CHUNK)], psgv.at[b], semg[b]).wait()
        pltpu.make_async_copy(
            ps_hbm.at[pl.ds(0, CHUNK)], pdgv.at[b], semg[b]).wait()
        pltpu.make_async_copy(
            pe_hbm.at[pl.ds(0, CHUNK // 8)], pev.at[b], semg[b]).wait()

    def wait_out(b):
        pltpu.make_async_copy(
            oute_hbm.at[pl.ds(0, CHUNK * E_OUT)], outv.at[b], semo[b]).wait()

    load_idx(0, 0)
    wait_idx(0)
    load_idx(1, 1)
    start_gathers(0, 0)

    def it_body(it, carry):
        for b in (0, 1):
            ci = 2 * it + b
            b1 = 1 - b
            wait_idx(b1)
            start_gathers(ci + 1, b1)
            wait_gathers(b)
            @pl.when(ci >= 2)
            def _():
                wait_out(b)

            def row(r, rcarry):
                pe_row = pev.at[b][
                    lax.shift_right_logical(r, 3),
                    pl.ds(lax.mul(lax.bitwise_and(r, 7), E_OUT), 16)]
                v = pe_row + psgv.at[b][r] + pdgv.at[b][r]
                outv.at[b][pl.ds(r * E_OUT, 16)] = jnp.maximum(v, 0.0)
                return rcarry

            lax.fori_loop(0, CHUNK, row, 0, unroll=4)

            valid = ((ci * NW + wid) < N_CHUNKS).astype(jnp.float32)
            vf = jnp.broadcast_to(valid, (16,))

            def grp(g, gcarry):
                flat = (lanes + g * 16) * E_OUT
                m = plsc.load_gather(outv.at[b], [flat])
                for c in range(1, E_OUT):
                    m = m + plsc.load_gather(outv.at[b], [flat + c])
                dv = dstv.at[b][pl.ds(g * 16, 16)]
                plsc.addupdate_scatter(sums_v, [dv], m * vf)
                plsc.addupdate_scatter(cnt_v, [dv], vf)
                return gcarry

            lax.fori_loop(0, CHUNK // 16, grp, 0, unroll=2)
            pltpu.async_copy(
                outv.at[b],
                oute_hbm.at[pl.ds(gbase(ci) * E_OUT, CHUNK * E_OUT)],
                semo[b])
            load_idx(ci + 2, b)
        return carry

    lax.fori_loop(0, ITERS // 2, it_body, 0)

    wait_gathers(0)
    wait_idx(1)
    wait_out(0)
    wait_out(1)

    pltpu.sync_copy(sums_v, psum_hbm.at[wid])
    pltpu.sync_copy(cnt_v, pcnt_hbm.at[wid])


_sc_edges = functools.partial(
    pl.kernel,
    out_type=[
        jax.ShapeDtypeStruct((N_EDGES * E_OUT,), jnp.float32),
        jax.ShapeDtypeStruct((NW, N_PAD), jnp.float32),
        jax.ShapeDtypeStruct((NW, N_PAD), jnp.float32),
    ],
    mesh=plsc.VectorSubcoreMesh(core_axis_name="c", subcore_axis_name="s"),
    compiler_params=pltpu.CompilerParams(
        use_tc_tiling_on_sc=False, needs_layout_passes=False),
    scratch_types=[
        pltpu.VMEM((2, CHUNK), jnp.int32),
        pltpu.VMEM((2, CHUNK), jnp.int32),
        pltpu.VMEM((2, CHUNK, E_OUT), jnp.float32),
        pltpu.VMEM((2, CHUNK, E_OUT), jnp.float32),
        pltpu.VMEM((2, CHUNK // 8, 128), jnp.float32),
        pltpu.VMEM((2, CHUNK * E_OUT), jnp.float32),
        pltpu.VMEM((SUM_BUF,), jnp.float32),
        pltpu.VMEM((SUM_BUF,), jnp.float32),
        pltpu.SemaphoreType.DMA,
        pltpu.SemaphoreType.DMA,
        pltpu.SemaphoreType.DMA,
        pltpu.SemaphoreType.DMA,
        pltpu.SemaphoreType.DMA,
        pltpu.SemaphoreType.DMA,
    ],
)(_sc_body)


def _newx_body(x_ref, psum_ref, pcnt_ref, w2a_ref, w2b_ref, b2_ref, out_ref):
    s = jnp.sum(psum_ref[...], axis=0) * (1.0 / E_OUT)
    c = jnp.sum(pcnt_ref[...], axis=0)
    attr = s / jnp.maximum(c, 1.0)
    acc = jnp.dot(x_ref[...], w2a_ref[...], preferred_element_type=jnp.float32)
    acc = acc + attr[:, None] * w2b_ref[...] + b2_ref[...]
    out_ref[...] = jnp.maximum(acc, 0.0)


def _node_update(x2d, psum, pcnt, w2a, w2b, b2):
    blk = 1024
    grid = N_PAD // blk
    return pl.pallas_call(
        _newx_body,
        grid=(grid,),
        in_specs=[
            pl.BlockSpec((blk, X_IN), lambda i: (i, 0)),
            pl.BlockSpec((NW, blk), lambda i: (0, i)),
            pl.BlockSpec((NW, blk), lambda i: (0, i)),
            pl.BlockSpec((X_IN, X_OUT), lambda i: (0, 0)),
            pl.BlockSpec((1, X_OUT), lambda i: (0, 0)),
            pl.BlockSpec((1, X_OUT), lambda i: (0, 0)),
        ],
        out_specs=pl.BlockSpec((blk, X_OUT), lambda i: (i, 0)),
        out_shape=jax.ShapeDtypeStruct((N_NODES, X_OUT), jnp.float32),
    )(x2d, psum, pcnt, w2a, w2b, b2)


def kernel(x, e, w1, b1, w2, b2, edge_index):
    x2d = x[0]
    e2d = e[0]
    src = edge_index[0]
    dst = edge_index[1]

    w1e = w1[:E_IN]
    w1s = w1[E_IN:E_IN + X_IN]
    w1d = w1[E_IN + X_IN:]
    b1r = b1.reshape(1, E_OUT)
    w2a = w2[:X_IN]
    w2b = w2[X_IN:].reshape(1, X_OUT)
    b2r = b2.reshape(1, X_OUT)

    e8 = e.reshape(N_EDGES // 8, 8 * E_IN)
    w8 = jnp.kron(jnp.eye(8, dtype=jnp.float32), w1e)
    b18 = jnp.tile(b1, 8).reshape(1, 8 * E_OUT)

    ps, pd = _node_tables(x2d, w1s, w1d)
    pe = _edge_table(e8, w8, b18)
    new_e_flat, psum, pcnt = _sc_edges(ps, pd, pe, src, dst)
    new_x = _node_update(x2d, psum, pcnt, w2a, w2b, b2r)

    return new_x[None], new_e_flat.reshape(N_EDGES, E_OUT)[None]


# speedup vs baseline: 5.1053x; 1.0364x over previous
import functools

import jax
import jax.numpy as jnp
from jax import lax
from jax.experimental import pallas as pl
from jax.experimental.pallas import tpu as pltpu
from jax.experimental.pallas import tpu_sc as plsc

N_NODES = 10000
N_EDGES = 160000
X_IN = 128
E_IN = 16
E_OUT = 16
X_OUT = 128

NC = 2
NS = 16
NW = NC * NS

CHUNK = 256
N_CHUNKS = N_EDGES // CHUNK
ITERS = -(-N_CHUNKS // NW)
N_PAD = 10240
SUM_BUF = N_PAD
OUT_BYTES = CHUNK * E_OUT * 4


def _tables_body(x_ref, ws_ref, wd_ref, ps_ref, pd_ref):
    xb = x_ref[...]
    ps_ref[...] = jnp.dot(xb, ws_ref[...], preferred_element_type=jnp.float32)
    pd_ref[...] = jnp.dot(xb, wd_ref[...], preferred_element_type=jnp.float32)


def _node_tables(x2d, w1s, w1d):
    blk = 1000
    grid = N_NODES // blk
    return pl.pallas_call(
        _tables_body,
        grid=(grid,),
        in_specs=[
            pl.BlockSpec((blk, X_IN), lambda i: (i, 0)),
            pl.BlockSpec((X_IN, E_OUT), lambda i: (0, 0)),
            pl.BlockSpec((X_IN, E_OUT), lambda i: (0, 0)),
        ],
        out_specs=[
            pl.BlockSpec((blk, E_OUT), lambda i: (i, 0)),
            pl.BlockSpec((blk, E_OUT), lambda i: (i, 0)),
        ],
        out_shape=[
            jax.ShapeDtypeStruct((N_NODES, E_OUT), jnp.float32),
            jax.ShapeDtypeStruct((N_NODES, E_OUT), jnp.float32),
        ],
    )(x2d, w1s, w1d)


def _pe_body(e_ref, we_ref, b1_ref, pe_ref):
    pe_ref[...] = (
        jnp.dot(e_ref[...], we_ref[...], preferred_element_type=jnp.float32)
        + b1_ref[...]
    )


def _edge_table(e8, w8, b18):
    blk = 2000
    grid = N_EDGES // 8 // blk
    return pl.pallas_call(
        _pe_body,
        grid=(grid,),
        in_specs=[
            pl.BlockSpec((blk, 128), lambda i: (i, 0)),
            pl.BlockSpec((128, 128), lambda i: (0, 0)),
            pl.BlockSpec((1, 128), lambda i: (0, 0)),
        ],
        out_specs=pl.BlockSpec((blk, 128), lambda i: (i, 0)),
        out_shape=jax.ShapeDtypeStruct((N_EDGES // 8, 128), jnp.float32),
    )(e8, w8, b18)


def _sc_body(ps_hbm, pd_hbm, pe_hbm, src_hbm, dst_hbm,
             oute_hbm, psum_hbm, pcnt_hbm,
             srcv, dstv, psgv, pdgv, pev, outv, sums_v, cnt_v,
             semi0, semi1, semg0, semg1, semo0, semo1):
    wid = lax.axis_index("s") * NC + lax.axis_index("c")
    semi = (semi0, semi1)
    semg = (semg0, semg1)
    semo = (semo0, semo1)
    zero16 = jnp.zeros((16,), jnp.float32)
    lanes = lax.iota(jnp.int32, 16)
    ones16 = jnp.full((16,), 1.0, jnp.float32)

    def zbody(i, carry):
        sums_v[pl.ds(i * 16, 16)] = zero16
        cnt_v[pl.ds(i * 16, 16)] = zero16
        return carry

    lax.fori_loop(0, SUM_BUF // 16, zbody, 0)

    def gchunk(ci):
        g = ci * NW + wid
        return jnp.minimum(g, N_CHUNKS - 1)

    def gbase(ci):
        return gchunk(ci) * CHUNK

    def load_idx(ci, b):
        base = gbase(ci)
        pltpu.async_copy(src_hbm.at[pl.ds(base, CHUNK)], srcv.at[b], semi[b])
        pltpu.async_copy(dst_hbm.at[pl.ds(base, CHUNK)], dstv.at[b], semi[b])

    def wait_idx(b):
        pltpu.make_async_copy(
            src_hbm.at[pl.ds(0, CHUNK)], srcv.at[b], semi[b]).wait()
        pltpu.make_async_copy(
            dst_hbm.at[pl.ds(0, CHUNK)], dstv.at[b], semi[b]).wait()

    def start_gathers(ci, b):
        for j in range(CHUNK // 128):
            pltpu.async_copy(
                ps_hbm.at[srcv.at[b].at[pl.ds(j * 128, 128)]],
                psgv.at[b].at[pl.ds(j * 128, 128)], semg[b])
            pltpu.async_copy(
                pd_hbm.at[dstv.at[b].at[pl.ds(j * 128, 128)]],
                pdgv.at[b].at[pl.ds(j * 128, 128)], semg[b])
        pltpu.async_copy(
            pe_hbm.at[pl.ds(gchunk(ci) * (CHUNK // 8), CHUNK // 8)],
            pev.at[b], semg[b])

    def wait_gathers(b):
        pltpu.make_async_copy(
            ps_hbm.at[pl.ds(0, CHUNK)], psgv.at[b], semg[b]).wait()
        pltpu.make_async_copy(
            ps_hbm.at[pl.ds(0, CHUNK)], pdgv.at[b], semg[b]).wait()
        pltpu.make_async_copy(
            pe_hbm.at[pl.ds(0, CHUNK // 8)], pev.at[b], semg[b]).wait()

    def wait_out(b):
        pltpu.make_async_copy(
            oute_hbm.at[pl.ds(0, CHUNK * E_OUT)], outv.at[b], semo[b]).wait()

    load_idx(0, 0)
    wait_idx(0)
    load_idx(1, 1)
    start_gathers(0, 0)

    def it_body(it, carry):
        for b in (0, 1):
            ci = 2 * it + b
            b1 = 1 - b
            wait_idx(b1)
            start_gathers(ci + 1, b1)
            wait_gathers(b)
            @pl.when(ci >= 2)
            def _():
                wait_out(b)

            def row8(q, rcarry):
                r0 = q * 8
                for k in range(8):
                    pe_row = pev.at[b][q, pl.ds(k * E_OUT, 16)]
                    v = pe_row + psgv.at[b][r0 + k] + pdgv.at[b][r0 + k]
                    outv.at[b][pl.ds((r0 + k) * E_OUT, 16)] = (
                        jnp.maximum(v, 0.0))
                return rcarry

            lax.fori_loop(0, CHUNK // 8, row8, 0, unroll=2)

            valid = ((ci * NW + wid) < N_CHUNKS).astype(jnp.float32)
            vf = jnp.broadcast_to(valid, (16,))

            def grp(g, gcarry):
                flat = (lanes + g * 16) * E_OUT
                m = plsc.load_gather(outv.at[b], [flat])
                for c in range(1, E_OUT):
                    m = m + plsc.load_gather(outv.at[b], [flat + c])
                dv = dstv.at[b][pl.ds(g * 16, 16)]
                plsc.addupdate_scatter(sums_v, [dv], m * vf)
                plsc.addupdate_scatter(cnt_v, [dv], vf)
                return gcarry

            lax.fori_loop(0, CHUNK // 16, grp, 0, unroll=2)
            pltpu.async_copy(
                outv.at[b],
                oute_hbm.at[pl.ds(gbase(ci) * E_OUT, CHUNK * E_OUT)],
                semo[b])
            load_idx(ci + 2, b)
        return carry

    lax.fori_loop(0, ITERS // 2, it_body, 0)

    wait_gathers(0)
    wait_idx(1)
    wait_out(0)
    wait_out(1)

    pltpu.sync_copy(sums_v, psum_hbm.at[wid])
    pltpu.sync_copy(cnt_v, pcnt_hbm.at[wid])


_sc_edges = functools.partial(
    pl.kernel,
    out_type=[
        jax.ShapeDtypeStruct((N_EDGES * E_OUT,), jnp.float32),
        jax.ShapeDtypeStruct((NW, N_PAD), jnp.float32),
        jax.ShapeDtypeStruct((NW, N_PAD), jnp.float32),
    ],
    mesh=plsc.VectorSubcoreMesh(core_axis_name="c", subcore_axis_name="s"),
    compiler_params=pltpu.CompilerParams(
        use_tc_tiling_on_sc=False, needs_layout_passes=False),
    scratch_types=[
        pltpu.VMEM((2, CHUNK), jnp.int32),
        pltpu.VMEM((2, CHUNK), jnp.int32),
        pltpu.VMEM((2, CHUNK, E_OUT), jnp.float32),
        pltpu.VMEM((2, CHUNK, E_OUT), jnp.float32),
        pltpu.VMEM((2, CHUNK // 8, 128), jnp.float32),
        pltpu.VMEM((2, CHUNK * E_OUT), jnp.float32),
        pltpu.VMEM((SUM_BUF,), jnp.float32),
        pltpu.VMEM((SUM_BUF,), jnp.float32),
        pltpu.SemaphoreType.DMA,
        pltpu.SemaphoreType.DMA,
        pltpu.SemaphoreType.DMA,
        pltpu.SemaphoreType.DMA,
        pltpu.SemaphoreType.DMA,
        pltpu.SemaphoreType.DMA,
    ],
)(_sc_body)


def _newx_body(x_ref, psum_ref, pcnt_ref, w2a_ref, w2b_ref, b2_ref, out_ref):
    s = jnp.sum(psum_ref[...], axis=0) * (1.0 / E_OUT)
    c = jnp.sum(pcnt_ref[...], axis=0)
    attr = s / jnp.maximum(c, 1.0)
    acc = jnp.dot(x_ref[...], w2a_ref[...], preferred_element_type=jnp.float32)
    acc = acc + attr[:, None] * w2b_ref[...] + b2_ref[...]
    out_ref[...] = jnp.maximum(acc, 0.0)


def _node_update(x2d, psum, pcnt, w2a, w2b, b2):
    blk = 1024
    grid = N_PAD // blk
    return pl.pallas_call(
        _newx_body,
        grid=(grid,),
        in_specs=[
            pl.BlockSpec((blk, X_IN), lambda i: (i, 0)),
            pl.BlockSpec((NW, blk), lambda i: (0, i)),
            pl.BlockSpec((NW, blk), lambda i: (0, i)),
            pl.BlockSpec((X_IN, X_OUT), lambda i: (0, 0)),
            pl.BlockSpec((1, X_OUT), lambda i: (0, 0)),
            pl.BlockSpec((1, X_OUT), lambda i: (0, 0)),
        ],
        out_specs=pl.BlockSpec((blk, X_OUT), lambda i: (i, 0)),
        out_shape=jax.ShapeDtypeStruct((N_NODES, X_OUT), jnp.float32),
    )(x2d, psum, pcnt, w2a, w2b, b2)


def kernel(x, e, w1, b1, w2, b2, edge_index):
    x2d = x[0]
    e2d = e[0]
    src = edge_index[0]
    dst = edge_index[1]

    w1e = w1[:E_IN]
    w1s = w1[E_IN:E_IN + X_IN]
    w1d = w1[E_IN + X_IN:]
    b1r = b1.reshape(1, E_OUT)
    w2a = w2[:X_IN]
    w2b = w2[X_IN:].reshape(1, X_OUT)
    b2r = b2.reshape(1, X_OUT)

    e8 = e.reshape(N_EDGES // 8, 8 * E_IN)
    w8 = jnp.kron(jnp.eye(8, dtype=jnp.float32), w1e)
    b18 = jnp.tile(b1, 8).reshape(1, 8 * E_OUT)

    ps, pd = _node_tables(x2d, w1s, w1d)
    pe = _edge_table(e8, w8, b18)
    new_e_flat, psum, pcnt = _sc_edges(ps, pd, pe, src, dst)
    new_x = _node_update(x2d, psum, pcnt, w2a, w2b, b2r)

    return new_x[None], new_e_flat.reshape(N_EDGES, E_OUT)[None]


# speedup vs baseline: 5.1893x; 1.0164x over previous
import functools

import jax
import jax.numpy as jnp
from jax import lax
from jax.experimental import pallas as pl
from jax.experimental.pallas import tpu as pltpu
from jax.experimental.pallas import tpu_sc as plsc

N_NODES = 10000
N_EDGES = 160000
X_IN = 128
E_IN = 16
E_OUT = 16
X_OUT = 128

NC = 2
NS = 16
NW = NC * NS

CHUNK = 512
N_CHUNKS = N_EDGES // CHUNK
ITERS = -(-N_CHUNKS // NW)
N_PAD = 10240
SUM_BUF = N_PAD
OUT_BYTES = CHUNK * E_OUT * 4


def _tables_body(x_ref, ws_ref, wd_ref, ps_ref, pd_ref):
    xb = x_ref[...]
    ps_ref[...] = jnp.dot(xb, ws_ref[...], preferred_element_type=jnp.float32)
    pd_ref[...] = jnp.dot(xb, wd_ref[...], preferred_element_type=jnp.float32)


def _node_tables(x2d, w1s, w1d):
    blk = 1000
    grid = N_NODES // blk
    return pl.pallas_call(
        _tables_body,
        grid=(grid,),
        in_specs=[
            pl.BlockSpec((blk, X_IN), lambda i: (i, 0)),
            pl.BlockSpec((X_IN, E_OUT), lambda i: (0, 0)),
            pl.BlockSpec((X_IN, E_OUT), lambda i: (0, 0)),
        ],
        out_specs=[
            pl.BlockSpec((blk, E_OUT), lambda i: (i, 0)),
            pl.BlockSpec((blk, E_OUT), lambda i: (i, 0)),
        ],
        out_shape=[
            jax.ShapeDtypeStruct((N_NODES, E_OUT), jnp.float32),
            jax.ShapeDtypeStruct((N_NODES, E_OUT), jnp.float32),
        ],
    )(x2d, w1s, w1d)


def _pe_body(e_ref, we_ref, b1_ref, pe_ref):
    pe_ref[...] = (
        jnp.dot(e_ref[...], we_ref[...], preferred_element_type=jnp.float32)
        + b1_ref[...]
    )


def _edge_table(e8, w8, b18):
    blk = 2000
    grid = N_EDGES // 8 // blk
    return pl.pallas_call(
        _pe_body,
        grid=(grid,),
        in_specs=[
            pl.BlockSpec((blk, 128), lambda i: (i, 0)),
            pl.BlockSpec((128, 128), lambda i: (0, 0)),
            pl.BlockSpec((1, 128), lambda i: (0, 0)),
        ],
        out_specs=pl.BlockSpec((blk, 128), lambda i: (i, 0)),
        out_shape=jax.ShapeDtypeStruct((N_EDGES // 8, 128), jnp.float32),
    )(e8, w8, b18)


def _sc_body(ps_hbm, pd_hbm, pe_hbm, src_hbm, dst_hbm,
             oute_hbm, psum_hbm, pcnt_hbm,
             srcv, dstv, psgv, pdgv, pev, outv, sums_v, cnt_v,
             semi0, semi1, semg0, semg1, semo0, semo1):
    wid = lax.axis_index("s") * NC + lax.axis_index("c")
    semi = (semi0, semi1)
    semg = (semg0, semg1)
    semo = (semo0, semo1)
    zero16 = jnp.zeros((16,), jnp.float32)
    lanes = lax.iota(jnp.int32, 16)
    ones16 = jnp.full((16,), 1.0, jnp.float32)

    def zbody(i, carry):
        sums_v[pl.ds(i * 16, 16)] = zero16
        cnt_v[pl.ds(i * 16, 16)] = zero16
        return carry

    lax.fori_loop(0, SUM_BUF // 16, zbody, 0)

    def gchunk(ci):
        g = ci * NW + wid
        return jnp.minimum(g, N_CHUNKS - 1)

    def gbase(ci):
        return gchunk(ci) * CHUNK

    def load_idx(ci, b):
        base = gbase(ci)
        pltpu.async_copy(src_hbm.at[pl.ds(base, CHUNK)], srcv.at[b], semi[b])
        pltpu.async_copy(dst_hbm.at[pl.ds(base, CHUNK)], dstv.at[b], semi[b])

    def wait_idx(b):
        pltpu.make_async_copy(
            src_hbm.at[pl.ds(0, CHUNK)], srcv.at[b], semi[b]).wait()
        pltpu.make_async_copy(
            dst_hbm.at[pl.ds(0, CHUNK)], dstv.at[b], semi[b]).wait()

    def start_gathers(ci, b):
        for j in range(CHUNK // 128):
            pltpu.async_copy(
                ps_hbm.at[srcv.at[b].at[pl.ds(j * 128, 128)]],
                psgv.at[b].at[pl.ds(j * 128, 128)], semg[b])
            pltpu.async_copy(
                pd_hbm.at[dstv.at[b].at[pl.ds(j * 128, 128)]],
                pdgv.at[b].at[pl.ds(j * 128, 128)], semg[b])
        pltpu.async_copy(
            pe_hbm.at[pl.ds(gchunk(ci) * (CHUNK // 8), CHUNK // 8)],
            pev.at[b], semg[b])

    def wait_gathers(b):
        pltpu.make_async_copy(
            ps_hbm.at[pl.ds(0, CHUNK)], psgv.at[b], semg[b]).wait()
        pltpu.make_async_copy(
            ps_hbm.at[pl.ds(0, CHUNK)], pdgv.at[b], semg[b]).wait()
        pltpu.make_async_copy(
            pe_hbm.at[pl.ds(0, CHUNK // 8)], pev.at[b], semg[b]).wait()

    def wait_out(b):
        pltpu.make_async_copy(
            oute_hbm.at[pl.ds(0, CHUNK * E_OUT)], outv.at[b], semo[b]).wait()

    load_idx(0, 0)
    wait_idx(0)
    load_idx(1, 1)
    start_gathers(0, 0)

    def it_body(it, carry):
        for b in (0, 1):
            ci = 2 * it + b
            b1 = 1 - b
            wait_idx(b1)
            start_gathers(ci + 1, b1)
            wait_gathers(b)
            @pl.when(ci >= 2)
            def _():
                wait_out(b)

            def row8(q, rcarry):
                r0 = q * 8
                for k in range(8):
                    pe_row = pev.at[b][q, pl.ds(k * E_OUT, 16)]
                    v = pe_row + psgv.at[b][r0 + k] + pdgv.at[b][r0 + k]
                    outv.at[b][pl.ds((r0 + k) * E_OUT, 16)] = (
                        jnp.maximum(v, 0.0))
                return rcarry

            lax.fori_loop(0, CHUNK // 8, row8, 0, unroll=2)

            valid = ((ci * NW + wid) < N_CHUNKS).astype(jnp.float32)
            vf = jnp.broadcast_to(valid, (16,))

            def grp(g, gcarry):
                flat = (lanes + g * 16) * E_OUT
                m = plsc.load_gather(outv.at[b], [flat])
                for c in range(1, E_OUT):
                    m = m + plsc.load_gather(outv.at[b], [flat + c])
                dv = dstv.at[b][pl.ds(g * 16, 16)]
                plsc.addupdate_scatter(sums_v, [dv], m * vf)
                plsc.addupdate_scatter(cnt_v, [dv], vf)
                return gcarry

            lax.fori_loop(0, CHUNK // 16, grp, 0, unroll=2)
            pltpu.async_copy(
                outv.at[b],
                oute_hbm.at[pl.ds(gbase(ci) * E_OUT, CHUNK * E_OUT)],
                semo[b])
            load_idx(ci + 2, b)
        return carry

    lax.fori_loop(0, ITERS // 2, it_body, 0)

    wait_gathers(0)
    wait_idx(1)
    wait_out(0)
    wait_out(1)

    pltpu.sync_copy(sums_v, psum_hbm.at[wid])
    pltpu.sync_copy(cnt_v, pcnt_hbm.at[wid])


_sc_edges = functools.partial(
    pl.kernel,
    out_type=[
        jax.ShapeDtypeStruct((N_EDGES * E_OUT,), jnp.float32),
        jax.ShapeDtypeStruct((NW, N_PAD), jnp.float32),
        jax.ShapeDtypeStruct((NW, N_PAD), jnp.float32),
    ],
    mesh=plsc.VectorSubcoreMesh(core_axis_name="c", subcore_axis_name="s"),
    compiler_params=pltpu.CompilerParams(
        use_tc_tiling_on_sc=False, needs_layout_passes=False),
    scratch_types=[
        pltpu.VMEM((2, CHUNK), jnp.int32),
        pltpu.VMEM((2, CHUNK), jnp.int32),
        pltpu.VMEM((2, CHUNK, E_OUT), jnp.float32),
        pltpu.VMEM((2, CHUNK, E_OUT), jnp.float32),
        pltpu.VMEM((2, CHUNK // 8, 128), jnp.float32),
        pltpu.VMEM((2, CHUNK * E_OUT), jnp.float32),
        pltpu.VMEM((SUM_BUF,), jnp.float32),
        pltpu.VMEM((SUM_BUF,), jnp.float32),
        pltpu.SemaphoreType.DMA,
        pltpu.SemaphoreType.DMA,
        pltpu.SemaphoreType.DMA,
        pltpu.SemaphoreType.DMA,
        pltpu.SemaphoreType.DMA,
        pltpu.SemaphoreType.DMA,
    ],
)(_sc_body)


def _newx_body(x_ref, psum_ref, pcnt_ref, w2a_ref, w2b_ref, b2_ref, out_ref):
    s = jnp.sum(psum_ref[...], axis=0) * (1.0 / E_OUT)
    c = jnp.sum(pcnt_ref[...], axis=0)
    attr = s / jnp.maximum(c, 1.0)
    acc = jnp.dot(x_ref[...], w2a_ref[...], preferred_element_type=jnp.float32)
    acc = acc + attr[:, None] * w2b_ref[...] + b2_ref[...]
    out_ref[...] = jnp.maximum(acc, 0.0)


def _node_update(x2d, psum, pcnt, w2a, w2b, b2):
    blk = 1024
    grid = N_PAD // blk
    return pl.pallas_call(
        _newx_body,
        grid=(grid,),
        in_specs=[
            pl.BlockSpec((blk, X_IN), lambda i: (i, 0)),
            pl.BlockSpec((NW, blk), lambda i: (0, i)),
            pl.BlockSpec((NW, blk), lambda i: (0, i)),
            pl.BlockSpec((X_IN, X_OUT), lambda i: (0, 0)),
            pl.BlockSpec((1, X_OUT), lambda i: (0, 0)),
            pl.BlockSpec((1, X_OUT), lambda i: (0, 0)),
        ],
        out_specs=pl.BlockSpec((blk, X_OUT), lambda i: (i, 0)),
        out_shape=jax.ShapeDtypeStruct((N_NODES, X_OUT), jnp.float32),
    )(x2d, psum, pcnt, w2a, w2b, b2)


def kernel(x, e, w1, b1, w2, b2, edge_index):
    x2d = x[0]
    e2d = e[0]
    src = edge_index[0]
    dst = edge_index[1]

    w1e = w1[:E_IN]
    w1s = w1[E_IN:E_IN + X_IN]
    w1d = w1[E_IN + X_IN:]
    b1r = b1.reshape(1, E_OUT)
    w2a = w2[:X_IN]
    w2b = w2[X_IN:].reshape(1, X_OUT)
    b2r = b2.reshape(1, X_OUT)

    e8 = e.reshape(N_EDGES // 8, 8 * E_IN)
    w8 = jnp.kron(jnp.eye(8, dtype=jnp.float32), w1e)
    b18 = jnp.tile(b1, 8).reshape(1, 8 * E_OUT)

    ps, pd = _node_tables(x2d, w1s, w1d)
    pe = _edge_table(e8, w8, b18)
    new_e_flat, psum, pcnt = _sc_edges(ps, pd, pe, src, dst)
    new_x = _node_update(x2d, psum, pcnt, w2a, w2b, b2r)

    return new_x[None], new_e_flat.reshape(N_EDGES, E_OUT)[None]


# speedup vs baseline: 5.1924x; 1.0006x over previous
import functools

import jax
import jax.numpy as jnp
from jax import lax
from jax.experimental import pallas as pl
from jax.experimental.pallas import tpu as pltpu
from jax.experimental.pallas import tpu_sc as plsc

N_NODES = 10000
N_EDGES = 160000
X_IN = 128
E_IN = 16
E_OUT = 16
X_OUT = 128

NC = 2
NS = 16
NW = NC * NS

CHUNK = 640
N_CHUNKS = N_EDGES // CHUNK
ITERS = -(-N_CHUNKS // NW)
N_PAD = 10240
SUM_BUF = N_PAD
OUT_BYTES = CHUNK * E_OUT * 4


def _tables_body(x_ref, ws_ref, wd_ref, ps_ref, pd_ref):
    xb = x_ref[...]
    ps_ref[...] = jnp.dot(xb, ws_ref[...], preferred_element_type=jnp.float32)
    pd_ref[...] = jnp.dot(xb, wd_ref[...], preferred_element_type=jnp.float32)


def _node_tables(x2d, w1s, w1d):
    blk = 1000
    grid = N_NODES // blk
    return pl.pallas_call(
        _tables_body,
        grid=(grid,),
        in_specs=[
            pl.BlockSpec((blk, X_IN), lambda i: (i, 0)),
            pl.BlockSpec((X_IN, E_OUT), lambda i: (0, 0)),
            pl.BlockSpec((X_IN, E_OUT), lambda i: (0, 0)),
        ],
        out_specs=[
            pl.BlockSpec((blk, E_OUT), lambda i: (i, 0)),
            pl.BlockSpec((blk, E_OUT), lambda i: (i, 0)),
        ],
        out_shape=[
            jax.ShapeDtypeStruct((N_NODES, E_OUT), jnp.float32),
            jax.ShapeDtypeStruct((N_NODES, E_OUT), jnp.float32),
        ],
    )(x2d, w1s, w1d)


def _pe_body(e_ref, we_ref, b1_ref, pe_ref):
    pe_ref[...] = (
        jnp.dot(e_ref[...], we_ref[...], preferred_element_type=jnp.float32)
        + b1_ref[...]
    )


def _edge_table(e8, w8, b18):
    blk = 2000
    grid = N_EDGES // 8 // blk
    return pl.pallas_call(
        _pe_body,
        grid=(grid,),
        in_specs=[
            pl.BlockSpec((blk, 128), lambda i: (i, 0)),
            pl.BlockSpec((128, 128), lambda i: (0, 0)),
            pl.BlockSpec((1, 128), lambda i: (0, 0)),
        ],
        out_specs=pl.BlockSpec((blk, 128), lambda i: (i, 0)),
        out_shape=jax.ShapeDtypeStruct((N_EDGES // 8, 128), jnp.float32),
    )(e8, w8, b18)


def _sc_body(ps_hbm, pd_hbm, pe_hbm, src_hbm, dst_hbm,
             oute_hbm, psum_hbm, pcnt_hbm,
             srcv, dstv, psgv, pdgv, pev, outv, sums_v, cnt_v,
             semi0, semi1, semg0, semg1, semo0, semo1):
    wid = lax.axis_index("s") * NC + lax.axis_index("c")
    semi = (semi0, semi1)
    semg = (semg0, semg1)
    semo = (semo0, semo1)
    zero16 = jnp.zeros((16,), jnp.float32)
    lanes = lax.iota(jnp.int32, 16)
    ones16 = jnp.full((16,), 1.0, jnp.float32)

    def zbody(i, carry):
        sums_v[pl.ds(i * 16, 16)] = zero16
        cnt_v[pl.ds(i * 16, 16)] = zero16
        return carry

    lax.fori_loop(0, SUM_BUF // 16, zbody, 0)

    def gchunk(ci):
        g = ci * NW + wid
        return jnp.minimum(g, N_CHUNKS - 1)

    def gbase(ci):
        return gchunk(ci) * CHUNK

    def load_idx(ci, b):
        base = gbase(ci)
        pltpu.async_copy(src_hbm.at[pl.ds(base, CHUNK)], srcv.at[b], semi[b])
        pltpu.async_copy(dst_hbm.at[pl.ds(base, CHUNK)], dstv.at[b], semi[b])

    def wait_idx(b):
        pltpu.make_async_copy(
            src_hbm.at[pl.ds(0, CHUNK)], srcv.at[b], semi[b]).wait()
        pltpu.make_async_copy(
            dst_hbm.at[pl.ds(0, CHUNK)], dstv.at[b], semi[b]).wait()

    def start_gathers(ci, b):
        for j in range(CHUNK // 128):
            pltpu.async_copy(
                ps_hbm.at[srcv.at[b].at[pl.ds(j * 128, 128)]],
                psgv.at[b].at[pl.ds(j * 128, 128)], semg[b])
            pltpu.async_copy(
                pd_hbm.at[dstv.at[b].at[pl.ds(j * 128, 128)]],
                pdgv.at[b].at[pl.ds(j * 128, 128)], semg[b])
        pltpu.async_copy(
            pe_hbm.at[pl.ds(gchunk(ci) * (CHUNK // 8), CHUNK // 8)],
            pev.at[b], semg[b])

    def wait_gathers(b):
        pltpu.make_async_copy(
            ps_hbm.at[pl.ds(0, CHUNK)], psgv.at[b], semg[b]).wait()
        pltpu.make_async_copy(
            ps_hbm.at[pl.ds(0, CHUNK)], pdgv.at[b], semg[b]).wait()
        pltpu.make_async_copy(
            pe_hbm.at[pl.ds(0, CHUNK // 8)], pev.at[b], semg[b]).wait()

    def wait_out(b):
        pltpu.make_async_copy(
            oute_hbm.at[pl.ds(0, CHUNK * E_OUT)], outv.at[b], semo[b]).wait()

    load_idx(0, 0)
    wait_idx(0)
    load_idx(1, 1)
    start_gathers(0, 0)

    def it_body(it, carry):
        for b in (0, 1):
            ci = 2 * it + b
            b1 = 1 - b
            wait_idx(b1)
            start_gathers(ci + 1, b1)
            wait_gathers(b)
            @pl.when(ci >= 2)
            def _():
                wait_out(b)

            def row8(q, rcarry):
                r0 = q * 8
                for k in range(8):
                    pe_row = pev.at[b][q, pl.ds(k * E_OUT, 16)]
                    v = pe_row + psgv.at[b][r0 + k] + pdgv.at[b][r0 + k]
                    outv.at[b][pl.ds((r0 + k) * E_OUT, 16)] = (
                        jnp.maximum(v, 0.0))
                return rcarry

            lax.fori_loop(0, CHUNK // 8, row8, 0, unroll=2)

            valid = ((ci * NW + wid) < N_CHUNKS).astype(jnp.float32)
            vf = jnp.broadcast_to(valid, (16,))

            def grp(g, gcarry):
                flat = (lanes + g * 16) * E_OUT
                m = plsc.load_gather(outv.at[b], [flat])
                for c in range(1, E_OUT):
                    m = m + plsc.load_gather(outv.at[b], [flat + c])
                dv = dstv.at[b][pl.ds(g * 16, 16)]
                plsc.addupdate_scatter(sums_v, [dv], m * vf)
                plsc.addupdate_scatter(cnt_v, [dv], vf)
                return gcarry

            lax.fori_loop(0, CHUNK // 16, grp, 0, unroll=2)
            pltpu.async_copy(
                outv.at[b],
                oute_hbm.at[pl.ds(gbase(ci) * E_OUT, CHUNK * E_OUT)],
                semo[b])
            load_idx(ci + 2, b)
        return carry

    lax.fori_loop(0, ITERS // 2, it_body, 0)

    wait_gathers(0)
    wait_idx(1)
    wait_out(0)
    wait_out(1)

    pltpu.sync_copy(sums_v, psum_hbm.at[wid])
    pltpu.sync_copy(cnt_v, pcnt_hbm.at[wid])


_sc_edges = functools.partial(
    pl.kernel,
    out_type=[
        jax.ShapeDtypeStruct((N_EDGES * E_OUT,), jnp.float32),
        jax.ShapeDtypeStruct((NW, N_PAD), jnp.float32),
        jax.ShapeDtypeStruct((NW, N_PAD), jnp.float32),
    ],
    mesh=plsc.VectorSubcoreMesh(core_axis_name="c", subcore_axis_name="s"),
    compiler_params=pltpu.CompilerParams(
        use_tc_tiling_on_sc=False, needs_layout_passes=False),
    scratch_types=[
        pltpu.VMEM((2, CHUNK), jnp.int32),
        pltpu.VMEM((2, CHUNK), jnp.int32),
        pltpu.VMEM((2, CHUNK, E_OUT), jnp.float32),
        pltpu.VMEM((2, CHUNK, E_OUT), jnp.float32),
        pltpu.VMEM((2, CHUNK // 8, 128), jnp.float32),
        pltpu.VMEM((2, CHUNK * E_OUT), jnp.float32),
        pltpu.VMEM((SUM_BUF,), jnp.float32),
        pltpu.VMEM((SUM_BUF,), jnp.float32),
        pltpu.SemaphoreType.DMA,
        pltpu.SemaphoreType.DMA,
        pltpu.SemaphoreType.DMA,
        pltpu.SemaphoreType.DMA,
        pltpu.SemaphoreType.DMA,
        pltpu.SemaphoreType.DMA,
    ],
)(_sc_body)


def _newx_body(x_ref, psum_ref, pcnt_ref, w2a_ref, w2b_ref, b2_ref, out_ref):
    s = jnp.sum(psum_ref[...], axis=0) * (1.0 / E_OUT)
    c = jnp.sum(pcnt_ref[...], axis=0)
    attr = s / jnp.maximum(c, 1.0)
    acc = jnp.dot(x_ref[...], w2a_ref[...], preferred_element_type=jnp.float32)
    acc = acc + attr[:, None] * w2b_ref[...] + b2_ref[...]
    out_ref[...] = jnp.maximum(acc, 0.0)


def _node_update(x2d, psum, pcnt, w2a, w2b, b2):
    blk = 1024
    grid = N_PAD // blk
    return pl.pallas_call(
        _newx_body,
        grid=(grid,),
        in_specs=[
            pl.BlockSpec((blk, X_IN), lambda i: (i, 0)),
            pl.BlockSpec((NW, blk), lambda i: (0, i)),
            pl.BlockSpec((NW, blk), lambda i: (0, i)),
            pl.BlockSpec((X_IN, X_OUT), lambda i: (0, 0)),
            pl.BlockSpec((1, X_OUT), lambda i: (0, 0)),
            pl.BlockSpec((1, X_OUT), lambda i: (0, 0)),
        ],
        out_specs=pl.BlockSpec((blk, X_OUT), lambda i: (i, 0)),
        out_shape=jax.ShapeDtypeStruct((N_NODES, X_OUT), jnp.float32),
    )(x2d, psum, pcnt, w2a, w2b, b2)


def kernel(x, e, w1, b1, w2, b2, edge_index):
    x2d = x[0]
    e2d = e[0]
    src = edge_index[0]
    dst = edge_index[1]

    w1e = w1[:E_IN]
    w1s = w1[E_IN:E_IN + X_IN]
    w1d = w1[E_IN + X_IN:]
    b1r = b1.reshape(1, E_OUT)
    w2a = w2[:X_IN]
    w2b = w2[X_IN:].reshape(1, X_OUT)
    b2r = b2.reshape(1, X_OUT)

    e8 = e.reshape(N_EDGES // 8, 8 * E_IN)
    w8 = jnp.kron(jnp.eye(8, dtype=jnp.float32), w1e)
    b18 = jnp.tile(b1, 8).reshape(1, 8 * E_OUT)

    ps, pd = _node_tables(x2d, w1s, w1d)
    pe = _edge_table(e8, w8, b18)
    new_e_flat, psum, pcnt = _sc_edges(ps, pd, pe, src, dst)
    new_x = _node_update(x2d, psum, pcnt, w2a, w2b, b2r)

    return new_x[None], new_e_flat.reshape(N_EDGES, E_OUT)[None]


# speedup vs baseline: 5.2122x; 1.0038x over previous
import functools

import jax
import jax.numpy as jnp
from jax import lax
from jax.experimental import pallas as pl
from jax.experimental.pallas import tpu as pltpu
from jax.experimental.pallas import tpu_sc as plsc

N_NODES = 10000
N_EDGES = 160000
X_IN = 128
E_IN = 16
E_OUT = 16
X_OUT = 128

NC = 2
NS = 16
NW = NC * NS

CHUNK = 640
N_CHUNKS = N_EDGES // CHUNK
ITERS = -(-N_CHUNKS // NW)
N_PAD = 10240
SUM_BUF = N_PAD
OUT_BYTES = CHUNK * E_OUT * 4


def _tables_body(x_ref, ws_ref, wd_ref, ps_ref, pd_ref):
    xb = x_ref[...]
    ps_ref[...] = jnp.dot(xb, ws_ref[...], preferred_element_type=jnp.float32)
    pd_ref[...] = jnp.dot(xb, wd_ref[...], preferred_element_type=jnp.float32)


def _node_tables(x2d, w1s, w1d):
    blk = 1000
    grid = N_NODES // blk
    return pl.pallas_call(
        _tables_body,
        grid=(grid,),
        in_specs=[
            pl.BlockSpec((blk, X_IN), lambda i: (i, 0)),
            pl.BlockSpec((X_IN, E_OUT), lambda i: (0, 0)),
            pl.BlockSpec((X_IN, E_OUT), lambda i: (0, 0)),
        ],
        out_specs=[
            pl.BlockSpec((blk, E_OUT), lambda i: (i, 0)),
            pl.BlockSpec((blk, E_OUT), lambda i: (i, 0)),
        ],
        out_shape=[
            jax.ShapeDtypeStruct((N_NODES, E_OUT), jnp.float32),
            jax.ShapeDtypeStruct((N_NODES, E_OUT), jnp.float32),
        ],
    )(x2d, w1s, w1d)


def _pe_body(e_ref, we_ref, b1_ref, pe_ref):
    pe_ref[...] = (
        jnp.dot(e_ref[...], we_ref[...], preferred_element_type=jnp.float32)
        + b1_ref[...]
    )


def _edge_table(e8, w8, b18):
    blk = 2000
    grid = N_EDGES // 8 // blk
    return pl.pallas_call(
        _pe_body,
        grid=(grid,),
        in_specs=[
            pl.BlockSpec((blk, 128), lambda i: (i, 0)),
            pl.BlockSpec((128, 128), lambda i: (0, 0)),
            pl.BlockSpec((1, 128), lambda i: (0, 0)),
        ],
        out_specs=pl.BlockSpec((blk, 128), lambda i: (i, 0)),
        out_shape=jax.ShapeDtypeStruct((N_EDGES // 8, 128), jnp.float32),
    )(e8, w8, b18)


def _sc_body(ps_hbm, pd_hbm, pe_hbm, src_hbm, dst_hbm,
             oute_hbm, psum_hbm, pcnt_hbm,
             srcv, dstv, psgv, pdgv, pev, outv, sums_v, cnt_v,
             semi0, semi1, semg0, semg1, semo0, semo1):
    wid = lax.axis_index("s") * NC + lax.axis_index("c")
    semi = (semi0, semi1)
    semg = (semg0, semg1)
    semo = (semo0, semo1)
    zero16 = jnp.zeros((16,), jnp.float32)
    lanes = lax.iota(jnp.int32, 16)
    ones16 = jnp.full((16,), 1.0, jnp.float32)

    def zbody(i, carry):
        sums_v[pl.ds(i * 16, 16)] = zero16
        cnt_v[pl.ds(i * 16, 16)] = zero16
        return carry

    lax.fori_loop(0, SUM_BUF // 16, zbody, 0)

    def gchunk(ci):
        g = ci * NW + wid
        return jnp.minimum(g, N_CHUNKS - 1)

    def gbase(ci):
        return gchunk(ci) * CHUNK

    def load_idx(ci, b):
        base = gbase(ci)
        pltpu.async_copy(src_hbm.at[pl.ds(base, CHUNK)], srcv.at[b], semi[b])
        pltpu.async_copy(dst_hbm.at[pl.ds(base, CHUNK)], dstv.at[b], semi[b])

    def wait_idx(b):
        pltpu.make_async_copy(
            src_hbm.at[pl.ds(0, CHUNK)], srcv.at[b], semi[b]).wait()
        pltpu.make_async_copy(
            dst_hbm.at[pl.ds(0, CHUNK)], dstv.at[b], semi[b]).wait()

    def start_gathers(ci, b):
        for j in range(CHUNK // 128):
            pltpu.async_copy(
                ps_hbm.at[srcv.at[b].at[pl.ds(j * 128, 128)]],
                psgv.at[b].at[pl.ds(j * 128, 128)], semg[b])
            pltpu.async_copy(
                pd_hbm.at[dstv.at[b].at[pl.ds(j * 128, 128)]],
                pdgv.at[b].at[pl.ds(j * 128, 128)], semg[b])
        pltpu.async_copy(
            pe_hbm.at[pl.ds(gchunk(ci) * (CHUNK // 8), CHUNK // 8)],
            pev.at[b], semg[b])

    def wait_gathers(b):
        pltpu.make_async_copy(
            ps_hbm.at[pl.ds(0, CHUNK)], psgv.at[b], semg[b]).wait()
        pltpu.make_async_copy(
            ps_hbm.at[pl.ds(0, CHUNK)], pdgv.at[b], semg[b]).wait()
        pltpu.make_async_copy(
            pe_hbm.at[pl.ds(0, CHUNK // 8)], pev.at[b], semg[b]).wait()

    def wait_out(b):
        pltpu.make_async_copy(
            oute_hbm.at[pl.ds(0, CHUNK * E_OUT)], outv.at[b], semo[b]).wait()

    load_idx(0, 0)
    wait_idx(0)
    load_idx(1, 1)
    start_gathers(0, 0)

    def it_body(it, carry):
        for b in (0, 1):
            ci = 2 * it + b
            b1 = 1 - b
            wait_idx(b1)
            start_gathers(ci + 1, b1)
            wait_gathers(b)
            @pl.when(ci >= 2)
            def _():
                wait_out(b)

            def row8(q, rcarry):
                r0 = q * 8
                for k in range(8):
                    pe_row = pev.at[b][q, pl.ds(k * E_OUT, 16)]
                    v = pe_row + psgv.at[b][r0 + k] + pdgv.at[b][r0 + k]
                    outv.at[b][pl.ds((r0 + k) * E_OUT, 16)] = (
                        jnp.maximum(v, 0.0))
                return rcarry

            lax.fori_loop(0, CHUNK // 8, row8, 0, unroll=2)

            valid = ((ci * NW + wid) < N_CHUNKS).astype(jnp.float32)
            vf = jnp.broadcast_to(valid, (16,))

            def grp(g, gcarry):
                flat = (lanes + g * 16) * E_OUT
                cols = [plsc.load_gather(outv.at[b], [flat + c] if c else [flat])
                        for c in range(E_OUT)]
                while len(cols) > 1:
                    cols = [cols[i] + cols[i + 1]
                            for i in range(0, len(cols), 2)]
                dv = dstv.at[b][pl.ds(g * 16, 16)]
                plsc.addupdate_scatter(sums_v, [dv], cols[0] * vf)
                plsc.addupdate_scatter(cnt_v, [dv], vf)
                return gcarry

            lax.fori_loop(0, CHUNK // 16, grp, 0, unroll=2)
            pltpu.async_copy(
                outv.at[b],
                oute_hbm.at[pl.ds(gbase(ci) * E_OUT, CHUNK * E_OUT)],
                semo[b])
            load_idx(ci + 2, b)
        return carry

    lax.fori_loop(0, ITERS // 2, it_body, 0)

    wait_gathers(0)
    wait_idx(1)
    wait_out(0)
    wait_out(1)

    pltpu.sync_copy(sums_v, psum_hbm.at[wid])
    pltpu.sync_copy(cnt_v, pcnt_hbm.at[wid])


_sc_edges = functools.partial(
    pl.kernel,
    out_type=[
        jax.ShapeDtypeStruct((N_EDGES * E_OUT,), jnp.float32),
        jax.ShapeDtypeStruct((NW, N_PAD), jnp.float32),
        jax.ShapeDtypeStruct((NW, N_PAD), jnp.float32),
    ],
    mesh=plsc.VectorSubcoreMesh(core_axis_name="c", subcore_axis_name="s"),
    compiler_params=pltpu.CompilerParams(
        use_tc_tiling_on_sc=False, needs_layout_passes=False),
    scratch_types=[
        pltpu.VMEM((2, CHUNK), jnp.int32),
        pltpu.VMEM((2, CHUNK), jnp.int32),
        pltpu.VMEM((2, CHUNK, E_OUT), jnp.float32),
        pltpu.VMEM((2, CHUNK, E_OUT), jnp.float32),
        pltpu.VMEM((2, CHUNK // 8, 128), jnp.float32),
        pltpu.VMEM((2, CHUNK * E_OUT), jnp.float32),
        pltpu.VMEM((SUM_BUF,), jnp.float32),
        pltpu.VMEM((SUM_BUF,), jnp.float32),
        pltpu.SemaphoreType.DMA,
        pltpu.SemaphoreType.DMA,
        pltpu.SemaphoreType.DMA,
        pltpu.SemaphoreType.DMA,
        pltpu.SemaphoreType.DMA,
        pltpu.SemaphoreType.DMA,
    ],
)(_sc_body)


def _newx_body(x_ref, psum_ref, pcnt_ref, w2a_ref, w2b_ref, b2_ref, out_ref):
    s = jnp.sum(psum_ref[...], axis=0) * (1.0 / E_OUT)
    c = jnp.sum(pcnt_ref[...], axis=0)
    attr = s / jnp.maximum(c, 1.0)
    acc = jnp.dot(x_ref[...], w2a_ref[...], preferred_element_type=jnp.float32)
    acc = acc + attr[:, None] * w2b_ref[...] + b2_ref[...]
    out_ref[...] = jnp.maximum(acc, 0.0)


def _node_update(x2d, psum, pcnt, w2a, w2b, b2):
    blk = 1024
    grid = N_PAD // blk
    return pl.pallas_call(
        _newx_body,
        grid=(grid,),
        in_specs=[
            pl.BlockSpec((blk, X_IN), lambda i: (i, 0)),
            pl.BlockSpec((NW, blk), lambda i: (0, i)),
            pl.BlockSpec((NW, blk), lambda i: (0, i)),
            pl.BlockSpec((X_IN, X_OUT), lambda i: (0, 0)),
            pl.BlockSpec((1, X_OUT), lambda i: (0, 0)),
            pl.BlockSpec((1, X_OUT), lambda i: (0, 0)),
        ],
        out_specs=pl.BlockSpec((blk, X_OUT), lambda i: (i, 0)),
        out_shape=jax.ShapeDtypeStruct((N_NODES, X_OUT), jnp.float32),
    )(x2d, psum, pcnt, w2a, w2b, b2)


def kernel(x, e, w1, b1, w2, b2, edge_index):
    x2d = x[0]
    e2d = e[0]
    src = edge_index[0]
    dst = edge_index[1]

    w1e = w1[:E_IN]
    w1s = w1[E_IN:E_IN + X_IN]
    w1d = w1[E_IN + X_IN:]
    b1r = b1.reshape(1, E_OUT)
    w2a = w2[:X_IN]
    w2b = w2[X_IN:].reshape(1, X_OUT)
    b2r = b2.reshape(1, X_OUT)

    e8 = e.reshape(N_EDGES // 8, 8 * E_IN)
    w8 = jnp.kron(jnp.eye(8, dtype=jnp.float32), w1e)
    b18 = jnp.tile(b1, 8).reshape(1, 8 * E_OUT)

    ps, pd = _node_tables(x2d, w1s, w1d)
    pe = _edge_table(e8, w8, b18)
    new_e_flat, psum, pcnt = _sc_edges(ps, pd, pe, src, dst)
    new_x = _node_update(x2d, psum, pcnt, w2a, w2b, b2r)

    return new_x[None], new_e_flat.reshape(N_EDGES, E_OUT)[None]


# speedup vs baseline: 5.2128x; 1.0001x over previous
import functools

import jax
import jax.numpy as jnp
from jax import lax
from jax.experimental import pallas as pl
from jax.experimental.pallas import tpu as pltpu
from jax.experimental.pallas import tpu_sc as plsc

N_NODES = 10000
N_EDGES = 160000
X_IN = 128
E_IN = 16
E_OUT = 16
X_OUT = 128

NC = 2
NS = 16
NW = NC * NS

CHUNK = 640
N_CHUNKS = N_EDGES // CHUNK
ITERS = -(-N_CHUNKS // NW)
N_PAD = 10240
SUM_BUF = N_PAD
OUT_BYTES = CHUNK * E_OUT * 4


def _tables_body(x_ref, ws_ref, wd_ref, ps_ref, pd_ref):
    xb = x_ref[...]
    ps_ref[...] = jnp.dot(xb, ws_ref[...], preferred_element_type=jnp.float32)
    pd_ref[...] = jnp.dot(xb, wd_ref[...], preferred_element_type=jnp.float32)


def _node_tables(x2d, w1s, w1d):
    blk = 1000
    grid = N_NODES // blk
    return pl.pallas_call(
        _tables_body,
        grid=(grid,),
        in_specs=[
            pl.BlockSpec((blk, X_IN), lambda i: (i, 0)),
            pl.BlockSpec((X_IN, E_OUT), lambda i: (0, 0)),
            pl.BlockSpec((X_IN, E_OUT), lambda i: (0, 0)),
        ],
        out_specs=[
            pl.BlockSpec((blk, E_OUT), lambda i: (i, 0)),
            pl.BlockSpec((blk, E_OUT), lambda i: (i, 0)),
        ],
        out_shape=[
            jax.ShapeDtypeStruct((N_NODES, E_OUT), jnp.float32),
            jax.ShapeDtypeStruct((N_NODES, E_OUT), jnp.float32),
        ],
    )(x2d, w1s, w1d)


def _pe_body(e_ref, we_ref, b1_ref, pe_ref):
    wrow = jnp.concatenate([we_ref[...]] * 8, axis=1)
    wrep = jnp.concatenate([wrow] * 8, axis=0)
    ri = lax.broadcasted_iota(jnp.int32, (128, 128), 0)
    ci = lax.broadcasted_iota(jnp.int32, (128, 128), 1)
    w8 = jnp.where((ri // E_IN) == (ci // E_OUT), wrep, 0.0)
    b18 = jnp.concatenate([b1_ref[...]] * 8, axis=1)
    pe_ref[...] = (
        jnp.dot(e_ref[...], w8, preferred_element_type=jnp.float32)
        + b18
    )


def _edge_table(e8, w1e, b1):
    blk = 2000
    grid = N_EDGES // 8 // blk
    return pl.pallas_call(
        _pe_body,
        grid=(grid,),
        in_specs=[
            pl.BlockSpec((blk, 128), lambda i: (i, 0)),
            pl.BlockSpec((E_IN, E_OUT), lambda i: (0, 0)),
            pl.BlockSpec((1, E_OUT), lambda i: (0, 0)),
        ],
        out_specs=pl.BlockSpec((blk, 128), lambda i: (i, 0)),
        out_shape=jax.ShapeDtypeStruct((N_EDGES // 8, 128), jnp.float32),
    )(e8, w1e, b1)


def _sc_body(ps_hbm, pd_hbm, pe_hbm, src_hbm, dst_hbm,
             oute_hbm, psum_hbm, pcnt_hbm,
             srcv, dstv, psgv, pdgv, pev, outv, sums_v, cnt_v,
             semi0, semi1, semg0, semg1, semo0, semo1):
    wid = lax.axis_index("s") * NC + lax.axis_index("c")
    semi = (semi0, semi1)
    semg = (semg0, semg1)
    semo = (semo0, semo1)
    zero16 = jnp.zeros((16,), jnp.float32)
    lanes = lax.iota(jnp.int32, 16)
    ones16 = jnp.full((16,), 1.0, jnp.float32)

    def zbody(i, carry):
        sums_v[pl.ds(i * 16, 16)] = zero16
        cnt_v[pl.ds(i * 16, 16)] = zero16
        return carry

    lax.fori_loop(0, SUM_BUF // 16, zbody, 0)

    def gchunk(ci):
        g = ci * NW + wid
        return jnp.minimum(g, N_CHUNKS - 1)

    def gbase(ci):
        return gchunk(ci) * CHUNK

    def load_idx(ci, b):
        base = gbase(ci)
        pltpu.async_copy(src_hbm.at[pl.ds(base, CHUNK)], srcv.at[b], semi[b])
        pltpu.async_copy(dst_hbm.at[pl.ds(base, CHUNK)], dstv.at[b], semi[b])

    def wait_idx(b):
        pltpu.make_async_copy(
            src_hbm.at[pl.ds(0, CHUNK)], srcv.at[b], semi[b]).wait()
        pltpu.make_async_copy(
            dst_hbm.at[pl.ds(0, CHUNK)], dstv.at[b], semi[b]).wait()

    def start_gathers(ci, b):
        for j in range(CHUNK // 128):
            pltpu.async_copy(
                ps_hbm.at[srcv.at[b].at[pl.ds(j * 128, 128)]],
                psgv.at[b].at[pl.ds(j * 128, 128)], semg[b])
            pltpu.async_copy(
                pd_hbm.at[dstv.at[b].at[pl.ds(j * 128, 128)]],
                pdgv.at[b].at[pl.ds(j * 128, 128)], semg[b])
        pltpu.async_copy(
            pe_hbm.at[pl.ds(gchunk(ci) * (CHUNK // 8), CHUNK // 8)],
            pev.at[b], semg[b])

    def wait_gathers(b):
        pltpu.make_async_copy(
            ps_hbm.at[pl.ds(0, CHUNK)], psgv.at[b], semg[b]).wait()
        pltpu.make_async_copy(
            ps_hbm.at[pl.ds(0, CHUNK)], pdgv.at[b], semg[b]).wait()
        pltpu.make_async_copy(
            pe_hbm.at[pl.ds(0, CHUNK // 8)], pev.at[b], semg[b]).wait()

    def wait_out(b):
        pltpu.make_async_copy(
            oute_hbm.at[pl.ds(0, CHUNK * E_OUT)], outv.at[b], semo[b]).wait()

    load_idx(0, 0)
    wait_idx(0)
    load_idx(1, 1)
    start_gathers(0, 0)

    def it_body(it, carry):
        for b in (0, 1):
            ci = 2 * it + b
            b1 = 1 - b
            wait_idx(b1)
            start_gathers(ci + 1, b1)
            wait_gathers(b)
            @pl.when(ci >= 2)
            def _():
                wait_out(b)

            def row8(q, rcarry):
                r0 = q * 8
                for k in range(8):
                    pe_row = pev.at[b][q, pl.ds(k * E_OUT, 16)]
                    v = pe_row + psgv.at[b][r0 + k] + pdgv.at[b][r0 + k]
                    outv.at[b][pl.ds((r0 + k) * E_OUT, 16)] = (
                        jnp.maximum(v, 0.0))
                return rcarry

            lax.fori_loop(0, CHUNK // 8, row8, 0, unroll=2)

            valid = ((ci * NW + wid) < N_CHUNKS).astype(jnp.float32)
            vf = jnp.broadcast_to(valid, (16,))

            def grp(g, gcarry):
                flat = (lanes + g * 16) * E_OUT
                cols = [plsc.load_gather(outv.at[b], [flat + c] if c else [flat])
                        for c in range(E_OUT)]
                while len(cols) > 1:
                    cols = [cols[i] + cols[i + 1]
                            for i in range(0, len(cols), 2)]
                dv = dstv.at[b][pl.ds(g * 16, 16)]
                plsc.addupdate_scatter(sums_v, [dv], cols[0] * vf)
                plsc.addupdate_scatter(cnt_v, [dv], vf)
                return gcarry

            lax.fori_loop(0, CHUNK // 16, grp, 0, unroll=2)
            pltpu.async_copy(
                outv.at[b],
                oute_hbm.at[pl.ds(gbase(ci) * E_OUT, CHUNK * E_OUT)],
                semo[b])
            load_idx(ci + 2, b)
        return carry

    lax.fori_loop(0, ITERS // 2, it_body, 0)

    wait_gathers(0)
    wait_idx(1)
    wait_out(0)
    wait_out(1)

    pltpu.sync_copy(sums_v, psum_hbm.at[wid])
    pltpu.sync_copy(cnt_v, pcnt_hbm.at[wid])


_sc_edges = functools.partial(
    pl.kernel,
    out_type=[
        jax.ShapeDtypeStruct((N_EDGES * E_OUT,), jnp.float32),
        jax.ShapeDtypeStruct((NW, N_PAD), jnp.float32),
        jax.ShapeDtypeStruct((NW, N_PAD), jnp.float32),
    ],
    mesh=plsc.VectorSubcoreMesh(core_axis_name="c", subcore_axis_name="s"),
    compiler_params=pltpu.CompilerParams(
        use_tc_tiling_on_sc=False, needs_layout_passes=False),
    scratch_types=[
        pltpu.VMEM((2, CHUNK), jnp.int32),
        pltpu.VMEM((2, CHUNK), jnp.int32),
        pltpu.VMEM((2, CHUNK, E_OUT), jnp.float32),
        pltpu.VMEM((2, CHUNK, E_OUT), jnp.float32),
        pltpu.VMEM((2, CHUNK // 8, 128), jnp.float32),
        pltpu.VMEM((2, CHUNK * E_OUT), jnp.float32),
        pltpu.VMEM((SUM_BUF,), jnp.float32),
        pltpu.VMEM((SUM_BUF,), jnp.float32),
        pltpu.SemaphoreType.DMA,
        pltpu.SemaphoreType.DMA,
        pltpu.SemaphoreType.DMA,
        pltpu.SemaphoreType.DMA,
        pltpu.SemaphoreType.DMA,
        pltpu.SemaphoreType.DMA,
    ],
)(_sc_body)


def _newx_body(x_ref, psum_ref, pcnt_ref, w2a_ref, w2b_ref, b2_ref, out_ref):
    s = jnp.sum(psum_ref[...], axis=0) * (1.0 / E_OUT)
    c = jnp.sum(pcnt_ref[...], axis=0)
    attr = s / jnp.maximum(c, 1.0)
    acc = jnp.dot(x_ref[...], w2a_ref[...], preferred_element_type=jnp.float32)
    acc = acc + attr[:, None] * w2b_ref[...] + b2_ref[...]
    out_ref[...] = jnp.maximum(acc, 0.0)


def _node_update(x2d, psum, pcnt, w2a, w2b, b2):
    blk = 1024
    grid = N_PAD // blk
    return pl.pallas_call(
        _newx_body,
        grid=(grid,),
        in_specs=[
            pl.BlockSpec((blk, X_IN), lambda i: (i, 0)),
            pl.BlockSpec((NW, blk), lambda i: (0, i)),
            pl.BlockSpec((NW, blk), lambda i: (0, i)),
            pl.BlockSpec((X_IN, X_OUT), lambda i: (0, 0)),
            pl.BlockSpec((1, X_OUT), lambda i: (0, 0)),
            pl.BlockSpec((1, X_OUT), lambda i: (0, 0)),
        ],
        out_specs=pl.BlockSpec((blk, X_OUT), lambda i: (i, 0)),
        out_shape=jax.ShapeDtypeStruct((N_NODES, X_OUT), jnp.float32),
    )(x2d, psum, pcnt, w2a, w2b, b2)


def kernel(x, e, w1, b1, w2, b2, edge_index):
    x2d = x[0]
    e2d = e[0]
    src = edge_index[0]
    dst = edge_index[1]

    w1e = w1[:E_IN]
    w1s = w1[E_IN:E_IN + X_IN]
    w1d = w1[E_IN + X_IN:]
    b1r = b1.reshape(1, E_OUT)
    w2a = w2[:X_IN]
    w2b = w2[X_IN:].reshape(1, X_OUT)
    b2r = b2.reshape(1, X_OUT)

    e8 = e.reshape(N_EDGES // 8, 8 * E_IN)

    ps, pd = _node_tables(x2d, w1s, w1d)
    pe = _edge_table(e8, w1e, b1r)
    new_e_flat, psum, pcnt = _sc_edges(ps, pd, pe, src, dst)
    new_x = _node_update(x2d, psum, pcnt, w2a, w2b, b2r)

    return new_x[None], new_e_flat.reshape(N_EDGES, E_OUT)[None]
